# baseline (speedup 1.0000x reference)
import functools

import jax
import jax.numpy as jnp
from jax import lax
from jax.experimental import pallas as pl
from jax.experimental.pallas import tpu as pltpu
from jax.experimental.pallas import tpu_sc as plsc

N = 10000
E = 160000
D = 256
HID = 256
ATT_DIM = 128
ATT_HEADS = 4
CLS = 128
NUM_CLASSES = 7

HALF = D // 2
EB = 128
NB_TOTAL = E // EB
NSUB = 16
TB_MAX = 80
TB_LAST = NB_TOTAL - TB_MAX * (NSUB - 1)
TB_LAST_PAD = 56
NB_PADDED = TB_MAX * (NSUB - 1) + TB_LAST_PAD
N_PAD = 10240
ROWS_PER_TILE = N_PAD // NSUB

EHALF = E // 2
DB_TOTAL = EHALF // EB
DB_BASE = DB_TOTAL // NSUB
DB_REM = DB_TOTAL - DB_BASE * NSUB

BN = 1024
GRID_N = (N + BN - 1) // BN


def _sc_segment_sum(h2, srcg, dst, zeros_hbm):
    mesh = plsc.VectorSubcoreMesh(core_axis_name="c", subcore_axis_name="s")

    @functools.partial(
        pl.kernel,
        out_type=jax.ShapeDtypeStruct((2 * N_PAD, HALF), jnp.float32),
        mesh=mesh,
        scratch_types=[
            pltpu.VMEM_SHARED((N_PAD, HALF), jnp.float32),
            pltpu.VMEM((TB_MAX * EB,), jnp.int32),
            pltpu.VMEM((2, EB), jnp.int32),
            pltpu.VMEM((2, EB, HALF), jnp.float32),
            pltpu.SemaphoreType.DMA,
            pltpu.SemaphoreType.DMA,
            pltpu.SemaphoreType.DMA,
            pltpu.SemaphoreType.DMA,
            pltpu.SemaphoreType.DMA,
            pltpu.SemaphoreType.DMA,
        ],
    )
    def k(h2_hbm, srcg_hbm, dst_hbm, zeros_hbm_, agg_out,
          agg_sh, idx_all, dst2_v, rows2, sem0, sem1, dem0, dem1, ssem0, ssem1):
        c = lax.axis_index("c")
        s = lax.axis_index("s")
        r0 = s * ROWS_PER_TILE
        pltpu.sync_copy(zeros_hbm_.at[pl.ds(r0, ROWS_PER_TILE)],
                        agg_sh.at[pl.ds(r0, ROWS_PER_TILE)])

        b0 = s * TB_MAX

        @pl.when(s < NSUB - 1)
        def _():
            pltpu.sync_copy(srcg_hbm.at[pl.ds(c * E + b0 * EB, TB_MAX * EB)],
                            idx_all)

        @pl.when(s == NSUB - 1)
        def _():
            pltpu.sync_copy(srcg_hbm.at[pl.ds(c * E + b0 * EB, TB_LAST * EB)],
                            idx_all.at[pl.ds(0, TB_LAST * EB)])

        plsc.subcore_barrier()

        nh = jnp.where(s < NSUB - 1, TB_MAX // 2, TB_LAST // 2)

        def gather(b, buf, sem):
            pltpu.async_copy(h2_hbm.at[idx_all.at[pl.ds(b * EB, EB)]],
                             rows2.at[buf], sem)

        def gwait(buf, sem):
            pltpu.make_async_copy(h2_hbm.at[pl.ds(0, EB)], rows2.at[buf], sem).wait()

        def dload(b, buf, sem):
            pltpu.async_copy(dst_hbm.at[pl.ds((b0 + b) * EB, EB)],
                             dst2_v.at[buf], sem)

        def dwait(buf, sem):
            pltpu.make_async_copy(dst_hbm.at[pl.ds(0, EB)], dst2_v.at[buf],
                                  sem).wait()

        gather(0, 0, sem0)
        gather(1, 1, sem1)
        dload(0, 0, dem0)
        dload(1, 1, dem1)

        def swait(buf, sem):
            pltpu.make_async_copy(h2_hbm.at[pl.ds(0, EB)], rows2.at[buf],
                                  sem).wait()

        def body(j, carry):
            b_even = 2 * j

            gwait(0, sem0)
            dwait(0, dem0)
            pltpu.async_copy(rows2.at[0], agg_sh.at[dst2_v.at[0]], ssem0,
                             add=True)

            gwait(1, sem1)
            dwait(1, dem1)
            pltpu.async_copy(rows2.at[1], agg_sh.at[dst2_v.at[1]], ssem1,
                             add=True)

            @pl.when(j < nh - 1)
            def _():
                swait(0, ssem0)
                gather(b_even + 2, 0, sem0)
                dload(b_even + 2, 0, dem0)
                swait(1, ssem1)
                gather(b_even + 3, 1, sem1)
                dload(b_even + 3, 1, dem1)

            return carry

        lax.fori_loop(0, nh, body, 0)
        swait(0, ssem0)
        swait(1, ssem1)

        plsc.subcore_barrier()
        pltpu.sync_copy(agg_sh.at[pl.ds(r0, ROWS_PER_TILE)],
                        agg_out.at[pl.ds(c * N_PAD + r0, ROWS_PER_TILE)])

    return k(h2, srcg, dst, zeros_hbm)


def _sc_degree(dst, zeros_hbm, ones_hbm):
    mesh = plsc.VectorSubcoreMesh(core_axis_name="c", subcore_axis_name="s")

    @functools.partial(
        pl.kernel,
        out_type=jax.ShapeDtypeStruct((2 * N_PAD, HALF), jnp.float32),
        mesh=mesh,
        scratch_types=[
            pltpu.VMEM_SHARED((N_PAD, HALF), jnp.float32),
            pltpu.VMEM((2, EB), jnp.int32),
            pltpu.VMEM((EB, HALF), jnp.float32),
            pltpu.SemaphoreType.DMA,
            pltpu.SemaphoreType.DMA,
        ],
    )
    def k(dst_hbm, zeros_hbm_, ones_hbm_, deg_out, deg_sh, dst2_v, ones_v,
          dem0, dem1):
        c = lax.axis_index("c")
        s = lax.axis_index("s")
        r0 = s * ROWS_PER_TILE
        pltpu.sync_copy(zeros_hbm_.at[pl.ds(r0, ROWS_PER_TILE)],
                        deg_sh.at[pl.ds(r0, ROWS_PER_TILE)])
        pltpu.sync_copy(ones_hbm_, ones_v)
        plsc.subcore_barrier()

        nb = jnp.where(s < 1, DB_BASE + 1, DB_BASE)
        dbase = c * DB_TOTAL + s * DB_BASE + jnp.minimum(s, 1)

        def dload(b, buf, sem):
            pltpu.async_copy(dst_hbm.at[pl.ds((dbase + b) * EB, EB)],
                             dst2_v.at[buf], sem)

        def dwait(buf, sem):
            pltpu.make_async_copy(dst_hbm.at[pl.ds(0, EB)], dst2_v.at[buf],
                                  sem).wait()

        dload(0, 0, dem0)
        dload(1, 1, dem1)

        def half(j, b, buf, sem):
            @pl.when(b < nb)
            def _():
                dwait(buf, sem)
                pltpu.sync_copy(ones_v, deg_sh.at[dst2_v.at[buf]], add=True)

                @pl.when(b + 2 < nb)
                def _():
                    dload(b + 2, buf, sem)

        def body(j, carry):
            half(j, 2 * j, 0, dem0)
            half(j, 2 * j + 1, 1, dem1)
            return carry

        lax.fori_loop(0, (DB_BASE + 2) // 2, body, 0)

        plsc.subcore_barrier()
        pltpu.sync_copy(deg_sh.at[pl.ds(r0, ROWS_PER_TILE)],
                        deg_out.at[pl.ds(c * N_PAD + r0, ROWS_PER_TILE)])

    return k(dst, zeros_hbm, ones_hbm)


def _layer_body(agg_lo, agg_hi, dega, degb, h, wlT, wrT, bl, g, b, out_ref):
    deg = dega[:, :1] + degb[:, :1]
    inv = 1.0 / jnp.maximum(deg, 1.0)
    mean = jnp.concatenate([agg_lo[...], agg_hi[...]], axis=1) * inv
    hb = h[...]
    out = (jnp.dot(mean, wlT[...], preferred_element_type=jnp.float32)
           + jnp.dot(hb, wrT[...], preferred_element_type=jnp.float32)
           + bl[...])
    nrm = jnp.maximum(jnp.sqrt(jnp.sum(out * out, axis=1, keepdims=True)), 1e-12)
    out = out / nrm
    mu = jnp.mean(out, axis=1, keepdims=True)
    var = jnp.mean((out - mu) ** 2, axis=1, keepdims=True)
    out = (out - mu) / jnp.sqrt(var + 1e-5) * g[...] + b[...]
    out = jnp.maximum(out, 0.0)
    out_ref[...] = out + hb


def _tc_layer(agg2, deg2, h, wlT, wrT, bl, g, b):
    full = lambda shape: pl.BlockSpec(shape, lambda i: (0,) * len(shape))
    nblk = N_PAD // BN
    return pl.pallas_call(
        _layer_body,
        grid=(GRID_N,),
        in_specs=[
            pl.BlockSpec((BN, HALF), lambda i: (i, 0)),
            pl.BlockSpec((BN, HALF), lambda i: (nblk + i, 0)),
            pl.BlockSpec((BN, HALF), lambda i: (i, 0)),
            pl.BlockSpec((BN, HALF), lambda i: (nblk + i, 0)),
            pl.BlockSpec((BN, HID), lambda i: (i, 0)),
            full((HID, HID)),
            full((HID, HID)),
            full((1, HID)),
            full((1, HID)),
            full((1, HID)),
        ],
        out_specs=pl.BlockSpec((BN, HID), lambda i: (i, 0)),
        out_shape=jax.ShapeDtypeStruct((N, HID), jnp.float32),
    )(agg2, agg2, deg2, deg2, h, wlT, wrT, bl, g, b)


def _scores_body(h, w1T, b1, w2bd, b2, s_ref, m_ref, m_scr):
    i = pl.program_id(0)
    hid = jnp.tanh(jnp.dot(h[...], w1T[...], preferred_element_type=jnp.float32)
                   + b1[...])
    s = jnp.dot(hid, w2bd[...], preferred_element_type=jnp.float32) + b2[...]
    s_ref[...] = s
    rows = i * BN + lax.broadcasted_iota(jnp.int32, s.shape, 0)
    sm = jnp.where(rows < N, s, -jnp.inf)
    bm = jnp.max(sm, axis=0, keepdims=True)

    @pl.when(i == 0)
    def _():
        m_scr[:1, :ATT_HEADS] = bm

    @pl.when(i > 0)
    def _():
        m_scr[:1, :ATT_HEADS] = jnp.maximum(m_scr[:1, :ATT_HEADS], bm)

    @pl.when(i == GRID_N - 1)
    def _():
        m_ref[...] = m_scr[:1, :ATT_HEADS]


def _tc_scores(h, w1T, b1, w2bd, b2):
    full = lambda shape: pl.BlockSpec(shape, lambda i: (0,) * len(shape))
    return pl.pallas_call(
        _scores_body,
        grid=(GRID_N,),
        in_specs=[
            pl.BlockSpec((BN, HID), lambda i: (i, 0)),
            full((HID, ATT_HEADS * ATT_DIM)),
            full((1, ATT_HEADS * ATT_DIM)),
            full((ATT_HEADS * ATT_DIM, ATT_HEADS)),
            full((1, ATT_HEADS)),
        ],
        out_specs=[
            pl.BlockSpec((BN, ATT_HEADS), lambda i: (i, 0)),
            pl.BlockSpec((1, ATT_HEADS), lambda i: (0, 0)),
        ],
        out_shape=[
            jax.ShapeDtypeStruct((N, ATT_HEADS), jnp.float32),
            jax.ShapeDtypeStruct((1, ATT_HEADS), jnp.float32),
        ],
        scratch_shapes=[pltpu.VMEM((8, 128), jnp.float32)],
    )(h, w1T, b1, w2bd, b2)


def _pool_body(s, m, h, e_ref, z_ref, p_ref, z_scr, p_scr):
    i = pl.program_id(0)
    e = jnp.exp(s[...] - m[...])
    rows = i * BN + lax.broadcasted_iota(jnp.int32, e.shape, 0)
    e = jnp.where(rows < N, e, 0.0)
    e_ref[...] = e
    zb = jnp.sum(e, axis=0, keepdims=True)
    hb = h[...]
    hrows = i * BN + lax.broadcasted_iota(jnp.int32, hb.shape, 0)
    hb = jnp.where(hrows < N, hb, 0.0)
    pb = lax.dot_general(e, hb, (((0,), (0,)), ((), ())),
                         preferred_element_type=jnp.float32)

    @pl.when(i == 0)
    def _():
        z_scr[:1, :ATT_HEADS] = zb
        p_scr[...] = pb

    @pl.when(i > 0)
    def _():
        z_scr[:1, :ATT_HEADS] = z_scr[:1, :ATT_HEADS] + zb
        p_scr[...] = p_scr[...] + pb

    @pl.when(i == GRID_N - 1)
    def _():
        z_ref[...] = z_scr[:1, :ATT_HEADS]
        p_ref[...] = p_scr[...]


def _tc_pool(s, m, h):
    full = lambda shape: pl.BlockSpec(shape, lambda i: (0,) * len(shape))
    return pl.pallas_call(
        _pool_body,
        grid=(GRID_N,),
        in_specs=[
            pl.BlockSpec((BN, ATT_HEADS), lambda i: (i, 0)),
            full((1, ATT_HEADS)),
            pl.BlockSpec((BN, HID), lambda i: (i, 0)),
        ],
        out_specs=[
            pl.BlockSpec((BN, ATT_HEADS), lambda i: (i, 0)),
            pl.BlockSpec((1, ATT_HEADS), lambda i: (0, 0)),
            pl.BlockSpec((ATT_HEADS, HID), lambda i: (0, 0)),
        ],
        out_shape=[
            jax.ShapeDtypeStruct((N, ATT_HEADS), jnp.float32),
            jax.ShapeDtypeStruct((1, ATT_HEADS), jnp.float32),
            jax.ShapeDtypeStruct((ATT_HEADS, HID), jnp.float32),
        ],
        scratch_shapes=[pltpu.VMEM((8, 128), jnp.float32),
                        pltpu.VMEM((ATT_HEADS, HID), jnp.float32)],
    )(s, m, h)


def _ln_row(t, g, b):
    mu = jnp.mean(t, axis=1, keepdims=True)
    var = jnp.mean((t - mu) ** 2, axis=1, keepdims=True)
    return (t - mu) / jnp.sqrt(var + 1e-5) * g + b


def _final_body(e, z, p, wc1T, bc1, g1, b1n, wc2T, bc2, g2, b2n, wc3T, bc3,
                a_ref, probs_ref):
    i = pl.program_id(0)
    a_ref[...] = e[...] * (1.0 / z[...])

    @pl.when(i == GRID_N - 1)
    def _():
        invz = (1.0 / z[...]).reshape(ATT_HEADS, 1)
        zagg = jnp.mean(p[...] * invz, axis=0, keepdims=True)
        t = jnp.dot(zagg, wc1T[...], preferred_element_type=jnp.float32) + bc1[...]
        t = jnp.maximum(_ln_row(t, g1[...], b1n[...]), 0.0)
        t = jnp.dot(t, wc2T[...], preferred_element_type=jnp.float32) + bc2[...]
        t = jnp.maximum(_ln_row(t, g2[...], b2n[...]), 0.0)
        logits = jnp.dot(t, wc3T[...], preferred_element_type=jnp.float32) + bc3[...]
        mx = jnp.max(logits, axis=1, keepdims=True)
        ex = jnp.exp(logits - mx)
        probs_ref[...] = ex / jnp.sum(ex, axis=1, keepdims=True)


def _tc_final(e, z, p, cw):
    full = lambda shape: pl.BlockSpec(shape, lambda i: (0,) * len(shape))
    return pl.pallas_call(
        _final_body,
        grid=(GRID_N,),
        in_specs=[
            pl.BlockSpec((BN, ATT_HEADS), lambda i: (i, 0)),
            full((1, ATT_HEADS)),
            full((ATT_HEADS, HID)),
            full((HID, CLS)), full((1, CLS)), full((1, CLS)), full((1, CLS)),
            full((CLS, CLS // 2)), full((1, CLS // 2)), full((1, CLS // 2)),
            full((1, CLS // 2)),
            full((CLS // 2, NUM_CLASSES)), full((1, NUM_CLASSES)),
        ],
        out_specs=[
            pl.BlockSpec((BN, ATT_HEADS), lambda i: (i, 0)),
            pl.BlockSpec((1, NUM_CLASSES), lambda i: (0, 0)),
        ],
        out_shape=[
            jax.ShapeDtypeStruct((N, ATT_HEADS), jnp.float32),
            jax.ShapeDtypeStruct((1, NUM_CLASSES), jnp.float32),
        ],
    )(e, z, p, *cw)


def kernel(x, edge_index, params):
    src = edge_index[0]
    dst = edge_index[1]
    srcg = jnp.concatenate([src * 2, src * 2 + 1])
    zeros_hbm = jnp.zeros((N_PAD, HALF), jnp.float32)
    ones_hbm = jnp.ones((EB, HALF), jnp.float32)

    deg2 = _sc_degree(dst, zeros_hbm, ones_hbm)

    h = x
    for lp in params['layers']:
        h2 = h.reshape(2 * N, HALF)
        agg2 = _sc_segment_sum(h2, srcg, dst, zeros_hbm)
        h = _tc_layer(agg2, deg2, h, lp['W_l'].T, lp['W_r'].T,
                      lp['b_l'].reshape(1, HID), lp['ln_g'].reshape(1, HID),
                      lp['ln_b'].reshape(1, HID))

    att = params['att']
    w1T = jnp.concatenate([ap['W1'] for ap in att], axis=0).T
    b1 = jnp.concatenate([ap['b1'] for ap in att]).reshape(1, -1)
    w2bd = jnp.zeros((ATT_HEADS * ATT_DIM, ATT_HEADS), jnp.float32)
    for hh, ap in enumerate(att):
        w2bd = w2bd.at[hh * ATT_DIM:(hh + 1) * ATT_DIM, hh].set(ap['W2'][0])
    b2 = jnp.stack([ap['b2'][0] for ap in att]).reshape(1, ATT_HEADS)

    s, m = _tc_scores(h, w1T, b1, w2bd, b2)
    e, z, p = _tc_pool(s, m, h)

    c = params['cls']
    cw = (c['W1'].T, c['b1'].reshape(1, -1), c['g1'].reshape(1, -1),
          c['b1n'].reshape(1, -1),
          c['W2'].T, c['b2'].reshape(1, -1), c['g2'].reshape(1, -1),
          c['b2n'].reshape(1, -1),
          c['W3'].T, c['b3'].reshape(1, -1))
    a, probs = _tc_final(e, z, p, cw)
    return probs[0], a


# speedup vs baseline: 1.1984x; 1.1984x over previous
import functools

import jax
import jax.numpy as jnp
from jax import lax
from jax.experimental import pallas as pl
from jax.experimental.pallas import tpu as pltpu
from jax.experimental.pallas import tpu_sc as plsc

N = 10000
E = 160000
D = 256
HID = 256
ATT_DIM = 128
ATT_HEADS = 4
CLS = 128
NUM_CLASSES = 7

HALF = D // 2
EB = 128
NB_TOTAL = E // EB
NSUB = 16
TB_MAX = 80
TB_LAST = NB_TOTAL - TB_MAX * (NSUB - 1)
TB_LAST_PAD = 56
NB_PADDED = TB_MAX * (NSUB - 1) + TB_LAST_PAD
N_PAD = 10240
ROWS_PER_TILE = N_PAD // NSUB

EHALF = E // 2
DB_TOTAL = EHALF // EB
DB_BASE = DB_TOTAL // NSUB
DB_REM = DB_TOTAL - DB_BASE * NSUB

BN = 1024
GRID_N = (N + BN - 1) // BN


def _sc_segment_sum(h2, srcg, dst, zeros_hbm):
    mesh = plsc.VectorSubcoreMesh(core_axis_name="c", subcore_axis_name="s")

    @functools.partial(
        pl.kernel,
        out_type=jax.ShapeDtypeStruct((2 * N_PAD, HALF), jnp.float32),
        mesh=mesh,
        scratch_types=[
            pltpu.VMEM_SHARED((N_PAD, HALF), jnp.float32),
            pltpu.VMEM((TB_MAX * EB,), jnp.int32),
            pltpu.VMEM((2, EB), jnp.int32),
            pltpu.VMEM((2, EB, HALF), jnp.float32),
            pltpu.SemaphoreType.DMA,
            pltpu.SemaphoreType.DMA,
            pltpu.SemaphoreType.DMA,
            pltpu.SemaphoreType.DMA,
        ],
    )
    def k(h2_hbm, srcg_hbm, dst_hbm, zeros_hbm_, agg_out,
          agg_sh, idx_all, dst2_v, rows2, sem0, sem1, dem0, dem1):
        c = lax.axis_index("c")
        s = lax.axis_index("s")
        r0 = s * ROWS_PER_TILE
        pltpu.sync_copy(zeros_hbm_.at[pl.ds(r0, ROWS_PER_TILE)],
                        agg_sh.at[pl.ds(r0, ROWS_PER_TILE)])

        b0 = s * TB_MAX

        @pl.when(s < NSUB - 1)
        def _():
            pltpu.sync_copy(srcg_hbm.at[pl.ds(c * E + b0 * EB, TB_MAX * EB)],
                            idx_all)

        @pl.when(s == NSUB - 1)
        def _():
            pltpu.sync_copy(srcg_hbm.at[pl.ds(c * E + b0 * EB, TB_LAST * EB)],
                            idx_all.at[pl.ds(0, TB_LAST * EB)])

        plsc.subcore_barrier()

        nh = jnp.where(s < NSUB - 1, TB_MAX // 2, TB_LAST // 2)

        def gather(b, buf, sem):
            pltpu.async_copy(h2_hbm.at[idx_all.at[pl.ds(b * EB, EB)]],
                             rows2.at[buf], sem)

        def gwait(buf, sem):
            pltpu.make_async_copy(h2_hbm.at[pl.ds(0, EB)], rows2.at[buf], sem).wait()

        def dload(b, buf, sem):
            pltpu.async_copy(dst_hbm.at[pl.ds((b0 + b) * EB, EB)],
                             dst2_v.at[buf], sem)

        def dwait(buf, sem):
            pltpu.make_async_copy(dst_hbm.at[pl.ds(0, EB)], dst2_v.at[buf],
                                  sem).wait()

        gather(0, 0, sem0)
        gather(1, 1, sem1)
        dload(0, 0, dem0)
        dload(1, 1, dem1)

        def body(j, carry):
            b_even = 2 * j

            gwait(0, sem0)
            dwait(0, dem0)
            pltpu.sync_copy(rows2.at[0], agg_sh.at[dst2_v.at[0]], add=True)

            @pl.when(j < nh - 1)
            def _():
                gather(b_even + 2, 0, sem0)
                dload(b_even + 2, 0, dem0)

            gwait(1, sem1)
            dwait(1, dem1)
            pltpu.sync_copy(rows2.at[1], agg_sh.at[dst2_v.at[1]], add=True)

            @pl.when(j < nh - 1)
            def _():
                gather(b_even + 3, 1, sem1)
                dload(b_even + 3, 1, dem1)

            return carry

        lax.fori_loop(0, nh, body, 0)

        plsc.subcore_barrier()
        pltpu.sync_copy(agg_sh.at[pl.ds(r0, ROWS_PER_TILE)],
                        agg_out.at[pl.ds(c * N_PAD + r0, ROWS_PER_TILE)])

    return k(h2, srcg, dst, zeros_hbm)


def _sc_degree(dst, zeros_hbm, ones_hbm):
    mesh = plsc.VectorSubcoreMesh(core_axis_name="c", subcore_axis_name="s")

    @functools.partial(
        pl.kernel,
        out_type=jax.ShapeDtypeStruct((2 * N_PAD, HALF), jnp.float32),
        mesh=mesh,
        scratch_types=[
            pltpu.VMEM_SHARED((N_PAD, HALF), jnp.float32),
            pltpu.VMEM((2, EB), jnp.int32),
            pltpu.VMEM((EB, HALF), jnp.float32),
            pltpu.SemaphoreType.DMA,
            pltpu.SemaphoreType.DMA,
        ],
    )
    def k(dst_hbm, zeros_hbm_, ones_hbm_, deg_out, deg_sh, dst2_v, ones_v,
          dem0, dem1):
        c = lax.axis_index("c")
        s = lax.axis_index("s")
        r0 = s * ROWS_PER_TILE
        pltpu.sync_copy(zeros_hbm_.at[pl.ds(r0, ROWS_PER_TILE)],
                        deg_sh.at[pl.ds(r0, ROWS_PER_TILE)])
        pltpu.sync_copy(ones_hbm_, ones_v)
        plsc.subcore_barrier()

        nb = jnp.where(s < 1, DB_BASE + 1, DB_BASE)
        dbase = c * DB_TOTAL + s * DB_BASE + jnp.minimum(s, 1)

        def dload(b, buf, sem):
            pltpu.async_copy(dst_hbm.at[pl.ds((dbase + b) * EB, EB)],
                             dst2_v.at[buf], sem)

        def dwait(buf, sem):
            pltpu.make_async_copy(dst_hbm.at[pl.ds(0, EB)], dst2_v.at[buf],
                                  sem).wait()

        dload(0, 0, dem0)
        dload(1, 1, dem1)

        def half(j, b, buf, sem):
            @pl.when(b < nb)
            def _():
                dwait(buf, sem)
                pltpu.sync_copy(ones_v, deg_sh.at[dst2_v.at[buf]], add=True)

                @pl.when(b + 2 < nb)
                def _():
                    dload(b + 2, buf, sem)

        def body(j, carry):
            half(j, 2 * j, 0, dem0)
            half(j, 2 * j + 1, 1, dem1)
            return carry

        lax.fori_loop(0, (DB_BASE + 2) // 2, body, 0)

        plsc.subcore_barrier()
        pltpu.sync_copy(deg_sh.at[pl.ds(r0, ROWS_PER_TILE)],
                        deg_out.at[pl.ds(c * N_PAD + r0, ROWS_PER_TILE)])

    return k(dst, zeros_hbm, ones_hbm)


def _layer_body(agg_lo, agg_hi, dega, degb, h, wlT, wrT, bl, g, b, out_ref):
    deg = dega[:, :1] + degb[:, :1]
    inv = 1.0 / jnp.maximum(deg, 1.0)
    mean = jnp.concatenate([agg_lo[...], agg_hi[...]], axis=1) * inv
    hb = h[...]
    out = (jnp.dot(mean, wlT[...], preferred_element_type=jnp.float32)
           + jnp.dot(hb, wrT[...], preferred_element_type=jnp.float32)
           + bl[...])
    nrm = jnp.maximum(jnp.sqrt(jnp.sum(out * out, axis=1, keepdims=True)), 1e-12)
    out = out / nrm
    mu = jnp.mean(out, axis=1, keepdims=True)
    var = jnp.mean((out - mu) ** 2, axis=1, keepdims=True)
    out = (out - mu) / jnp.sqrt(var + 1e-5) * g[...] + b[...]
    out = jnp.maximum(out, 0.0)
    out_ref[...] = out + hb


def _tc_layer(agg2, deg2, h, wlT, wrT, bl, g, b):
    full = lambda shape: pl.BlockSpec(shape, lambda i: (0,) * len(shape))
    nblk = N_PAD // BN
    return pl.pallas_call(
        _layer_body,
        grid=(GRID_N,),
        in_specs=[
            pl.BlockSpec((BN, HALF), lambda i: (i, 0)),
            pl.BlockSpec((BN, HALF), lambda i: (nblk + i, 0)),
            pl.BlockSpec((BN, HALF), lambda i: (i, 0)),
            pl.BlockSpec((BN, HALF), lambda i: (nblk + i, 0)),
            pl.BlockSpec((BN, HID), lambda i: (i, 0)),
            full((HID, HID)),
            full((HID, HID)),
            full((1, HID)),
            full((1, HID)),
            full((1, HID)),
        ],
        out_specs=pl.BlockSpec((BN, HID), lambda i: (i, 0)),
        out_shape=jax.ShapeDtypeStruct((N, HID), jnp.float32),
    )(agg2, agg2, deg2, deg2, h, wlT, wrT, bl, g, b)


def _scores_body(h, w1T, b1, w2bd, b2, s_ref, m_ref, m_scr):
    i = pl.program_id(0)
    hid = jnp.tanh(jnp.dot(h[...], w1T[...], preferred_element_type=jnp.float32)
                   + b1[...])
    s = jnp.dot(hid, w2bd[...], preferred_element_type=jnp.float32) + b2[...]
    s_ref[...] = s
    rows = i * BN + lax.broadcasted_iota(jnp.int32, s.shape, 0)
    sm = jnp.where(rows < N, s, -jnp.inf)
    bm = jnp.max(sm, axis=0, keepdims=True)

    @pl.when(i == 0)
    def _():
        m_scr[:1, :ATT_HEADS] = bm

    @pl.when(i > 0)
    def _():
        m_scr[:1, :ATT_HEADS] = jnp.maximum(m_scr[:1, :ATT_HEADS], bm)

    @pl.when(i == GRID_N - 1)
    def _():
        m_ref[...] = m_scr[:1, :ATT_HEADS]


def _tc_scores(h, w1T, b1, w2bd, b2):
    full = lambda shape: pl.BlockSpec(shape, lambda i: (0,) * len(shape))
    return pl.pallas_call(
        _scores_body,
        grid=(GRID_N,),
        in_specs=[
            pl.BlockSpec((BN, HID), lambda i: (i, 0)),
            full((HID, ATT_HEADS * ATT_DIM)),
            full((1, ATT_HEADS * ATT_DIM)),
            full((ATT_HEADS * ATT_DIM, ATT_HEADS)),
            full((1, ATT_HEADS)),
        ],
        out_specs=[
            pl.BlockSpec((BN, ATT_HEADS), lambda i: (i, 0)),
            pl.BlockSpec((1, ATT_HEADS), lambda i: (0, 0)),
        ],
        out_shape=[
            jax.ShapeDtypeStruct((N, ATT_HEADS), jnp.float32),
            jax.ShapeDtypeStruct((1, ATT_HEADS), jnp.float32),
        ],
        scratch_shapes=[pltpu.VMEM((8, 128), jnp.float32)],
    )(h, w1T, b1, w2bd, b2)


def _pool_body(s, m, h, e_ref, z_ref, p_ref, z_scr, p_scr):
    i = pl.program_id(0)
    e = jnp.exp(s[...] - m[...])
    rows = i * BN + lax.broadcasted_iota(jnp.int32, e.shape, 0)
    e = jnp.where(rows < N, e, 0.0)
    e_ref[...] = e
    zb = jnp.sum(e, axis=0, keepdims=True)
    hb = h[...]
    hrows = i * BN + lax.broadcasted_iota(jnp.int32, hb.shape, 0)
    hb = jnp.where(hrows < N, hb, 0.0)
    pb = lax.dot_general(e, hb, (((0,), (0,)), ((), ())),
                         preferred_element_type=jnp.float32)

    @pl.when(i == 0)
    def _():
        z_scr[:1, :ATT_HEADS] = zb
        p_scr[...] = pb

    @pl.when(i > 0)
    def _():
        z_scr[:1, :ATT_HEADS] = z_scr[:1, :ATT_HEADS] + zb
        p_scr[...] = p_scr[...] + pb

    @pl.when(i == GRID_N - 1)
    def _():
        z_ref[...] = z_scr[:1, :ATT_HEADS]
        p_ref[...] = p_scr[...]


def _tc_pool(s, m, h):
    full = lambda shape: pl.BlockSpec(shape, lambda i: (0,) * len(shape))
    return pl.pallas_call(
        _pool_body,
        grid=(GRID_N,),
        in_specs=[
            pl.BlockSpec((BN, ATT_HEADS), lambda i: (i, 0)),
            full((1, ATT_HEADS)),
            pl.BlockSpec((BN, HID), lambda i: (i, 0)),
        ],
        out_specs=[
            pl.BlockSpec((BN, ATT_HEADS), lambda i: (i, 0)),
            pl.BlockSpec((1, ATT_HEADS), lambda i: (0, 0)),
            pl.BlockSpec((ATT_HEADS, HID), lambda i: (0, 0)),
        ],
        out_shape=[
            jax.ShapeDtypeStruct((N, ATT_HEADS), jnp.float32),
            jax.ShapeDtypeStruct((1, ATT_HEADS), jnp.float32),
            jax.ShapeDtypeStruct((ATT_HEADS, HID), jnp.float32),
        ],
        scratch_shapes=[pltpu.VMEM((8, 128), jnp.float32),
                        pltpu.VMEM((ATT_HEADS, HID), jnp.float32)],
    )(s, m, h)


def _ln_row(t, g, b):
    mu = jnp.mean(t, axis=1, keepdims=True)
    var = jnp.mean((t - mu) ** 2, axis=1, keepdims=True)
    return (t - mu) / jnp.sqrt(var + 1e-5) * g + b


def _final_body(e, z, p, wc1T, bc1, g1, b1n, wc2T, bc2, g2, b2n, wc3T, bc3,
                a_ref, probs_ref):
    i = pl.program_id(0)
    a_ref[...] = e[...] * (1.0 / z[...])

    @pl.when(i == GRID_N - 1)
    def _():
        invz = (1.0 / z[...]).reshape(ATT_HEADS, 1)
        zagg = jnp.mean(p[...] * invz, axis=0, keepdims=True)
        t = jnp.dot(zagg, wc1T[...], preferred_element_type=jnp.float32) + bc1[...]
        t = jnp.maximum(_ln_row(t, g1[...], b1n[...]), 0.0)
        t = jnp.dot(t, wc2T[...], preferred_element_type=jnp.float32) + bc2[...]
        t = jnp.maximum(_ln_row(t, g2[...], b2n[...]), 0.0)
        logits = jnp.dot(t, wc3T[...], preferred_element_type=jnp.float32) + bc3[...]
        mx = jnp.max(logits, axis=1, keepdims=True)
        ex = jnp.exp(logits - mx)
        probs_ref[...] = ex / jnp.sum(ex, axis=1, keepdims=True)


def _tc_final(e, z, p, cw):
    full = lambda shape: pl.BlockSpec(shape, lambda i: (0,) * len(shape))
    return pl.pallas_call(
        _final_body,
        grid=(GRID_N,),
        in_specs=[
            pl.BlockSpec((BN, ATT_HEADS), lambda i: (i, 0)),
            full((1, ATT_HEADS)),
            full((ATT_HEADS, HID)),
            full((HID, CLS)), full((1, CLS)), full((1, CLS)), full((1, CLS)),
            full((CLS, CLS // 2)), full((1, CLS // 2)), full((1, CLS // 2)),
            full((1, CLS // 2)),
            full((CLS // 2, NUM_CLASSES)), full((1, NUM_CLASSES)),
        ],
        out_specs=[
            pl.BlockSpec((BN, ATT_HEADS), lambda i: (i, 0)),
            pl.BlockSpec((1, NUM_CLASSES), lambda i: (0, 0)),
        ],
        out_shape=[
            jax.ShapeDtypeStruct((N, ATT_HEADS), jnp.float32),
            jax.ShapeDtypeStruct((1, NUM_CLASSES), jnp.float32),
        ],
    )(e, z, p, *cw)


def kernel(x, edge_index, params):
    src = edge_index[0]
    dst = edge_index[1]
    srcg = jnp.concatenate([src * 2, src * 2 + 1])
    zeros_hbm = jnp.zeros((N_PAD, HALF), jnp.float32)
    ones_hbm = jnp.ones((EB, HALF), jnp.float32)

    deg2 = _sc_degree(dst, zeros_hbm, ones_hbm)

    h = x
    for lp in params['layers']:
        h2 = h.reshape(2 * N, HALF)
        agg2 = _sc_segment_sum(h2, srcg, dst, zeros_hbm)
        h = _tc_layer(agg2, deg2, h, lp['W_l'].T, lp['W_r'].T,
                      lp['b_l'].reshape(1, HID), lp['ln_g'].reshape(1, HID),
                      lp['ln_b'].reshape(1, HID))

    att = params['att']
    w1T = jnp.concatenate([ap['W1'] for ap in att], axis=0).T
    b1 = jnp.concatenate([ap['b1'] for ap in att]).reshape(1, -1)
    w2bd = jnp.zeros((ATT_HEADS * ATT_DIM, ATT_HEADS), jnp.float32)
    for hh, ap in enumerate(att):
        w2bd = w2bd.at[hh * ATT_DIM:(hh + 1) * ATT_DIM, hh].set(ap['W2'][0])
    b2 = jnp.stack([ap['b2'][0] for ap in att]).reshape(1, ATT_HEADS)

    s, m = _tc_scores(h, w1T, b1, w2bd, b2)
    e, z, p = _tc_pool(s, m, h)

    c = params['cls']
    cw = (c['W1'].T, c['b1'].reshape(1, -1), c['g1'].reshape(1, -1),
          c['b1n'].reshape(1, -1),
          c['W2'].T, c['b2'].reshape(1, -1), c['g2'].reshape(1, -1),
          c['b2n'].reshape(1, -1),
          c['W3'].T, c['b3'].reshape(1, -1))
    a, probs = _tc_final(e, z, p, cw)
    return probs[0], a


# speedup vs baseline: 1.2221x; 1.0198x over previous
import functools

import jax
import jax.numpy as jnp
from jax import lax
from jax.experimental import pallas as pl
from jax.experimental.pallas import tpu as pltpu
from jax.experimental.pallas import tpu_sc as plsc

N = 10000
E = 160000
D = 256
HID = 256
ATT_DIM = 128
ATT_HEADS = 4
CLS = 128
NUM_CLASSES = 7

HALF = D // 2
EB = 128
NB_TOTAL = E // EB
NSUB = 16
TB_MAX = 80
TB_LAST = NB_TOTAL - TB_MAX * (NSUB - 1)
TB_LAST_PAD = 56
NB_PADDED = TB_MAX * (NSUB - 1) + TB_LAST_PAD
N_PAD = 10240
ROWS_PER_TILE = N_PAD // NSUB

EHALF = E // 2
DB_TOTAL = EHALF // EB
DB_BASE = DB_TOTAL // NSUB
DB_REM = DB_TOTAL - DB_BASE * NSUB

BN = 1024
GRID_N = (N + BN - 1) // BN


def _sc_segment_sum(h2, srcg, dst, zeros_hbm):
    mesh = plsc.VectorSubcoreMesh(core_axis_name="c", subcore_axis_name="s")

    @functools.partial(
        pl.kernel,
        out_type=jax.ShapeDtypeStruct((2 * N_PAD, HALF), jnp.float32),
        mesh=mesh,
        scratch_types=[
            pltpu.VMEM_SHARED((N_PAD, HALF), jnp.float32),
            pltpu.VMEM((TB_MAX * EB,), jnp.int32),
            pltpu.VMEM((2, EB), jnp.int32),
            pltpu.VMEM((2, EB, HALF), jnp.float32),
            pltpu.SemaphoreType.DMA,
            pltpu.SemaphoreType.DMA,
            pltpu.SemaphoreType.DMA,
            pltpu.SemaphoreType.DMA,
        ],
    )
    def k(h2_hbm, srcg_hbm, dst_hbm, zeros_hbm_, agg_out,
          agg_sh, idx_all, dst2_v, rows2, sem0, sem1, dem0, dem1):
        c = lax.axis_index("c")
        s = lax.axis_index("s")
        r0 = s * ROWS_PER_TILE
        pltpu.sync_copy(zeros_hbm_.at[pl.ds(r0, ROWS_PER_TILE)],
                        agg_sh.at[pl.ds(r0, ROWS_PER_TILE)])

        b0 = s * TB_MAX

        @pl.when(s < NSUB - 1)
        def _():
            pltpu.sync_copy(srcg_hbm.at[pl.ds(c * E + b0 * EB, TB_MAX * EB)],
                            idx_all)

        @pl.when(s == NSUB - 1)
        def _():
            pltpu.sync_copy(srcg_hbm.at[pl.ds(c * E + b0 * EB, TB_LAST * EB)],
                            idx_all.at[pl.ds(0, TB_LAST * EB)])

        plsc.subcore_barrier()

        nh = jnp.where(s < NSUB - 1, TB_MAX // 2, TB_LAST // 2)

        def gather(b, buf, sem):
            pltpu.async_copy(h2_hbm.at[idx_all.at[pl.ds(b * EB, EB)]],
                             rows2.at[buf], sem)

        def gwait(buf, sem):
            pltpu.make_async_copy(h2_hbm.at[pl.ds(0, EB)], rows2.at[buf], sem).wait()

        def dload(b, buf, sem):
            pltpu.async_copy(dst_hbm.at[pl.ds((b0 + b) * EB, EB)],
                             dst2_v.at[buf], sem)

        def dwait(buf, sem):
            pltpu.make_async_copy(dst_hbm.at[pl.ds(0, EB)], dst2_v.at[buf],
                                  sem).wait()

        gather(0, 0, sem0)
        gather(1, 1, sem1)
        dload(0, 0, dem0)
        dload(1, 1, dem1)

        def body(j, carry):
            b_even = 2 * j

            gwait(0, sem0)
            dwait(0, dem0)
            pltpu.sync_copy(rows2.at[0], agg_sh.at[dst2_v.at[0]], add=True)

            @pl.when(j < nh - 1)
            def _():
                gather(b_even + 2, 0, sem0)
                dload(b_even + 2, 0, dem0)

            gwait(1, sem1)
            dwait(1, dem1)
            pltpu.sync_copy(rows2.at[1], agg_sh.at[dst2_v.at[1]], add=True)

            @pl.when(j < nh - 1)
            def _():
                gather(b_even + 3, 1, sem1)
                dload(b_even + 3, 1, dem1)

            return carry

        lax.fori_loop(0, nh, body, 0)

        plsc.subcore_barrier()
        pltpu.sync_copy(agg_sh.at[pl.ds(r0, ROWS_PER_TILE)],
                        agg_out.at[pl.ds(c * N_PAD + r0, ROWS_PER_TILE)])

    return k(h2, srcg, dst, zeros_hbm)


def _sc_degree(dst, zeros_hbm, ones_hbm):
    mesh = plsc.VectorSubcoreMesh(core_axis_name="c", subcore_axis_name="s")

    @functools.partial(
        pl.kernel,
        out_type=jax.ShapeDtypeStruct((2 * N_PAD, HALF), jnp.float32),
        mesh=mesh,
        scratch_types=[
            pltpu.VMEM_SHARED((N_PAD, HALF), jnp.float32),
            pltpu.VMEM((2, EB), jnp.int32),
            pltpu.VMEM((EB, HALF), jnp.float32),
            pltpu.SemaphoreType.DMA,
            pltpu.SemaphoreType.DMA,
        ],
    )
    def k(dst_hbm, zeros_hbm_, ones_hbm_, deg_out, deg_sh, dst2_v, ones_v,
          dem0, dem1):
        c = lax.axis_index("c")
        s = lax.axis_index("s")
        r0 = s * ROWS_PER_TILE
        pltpu.sync_copy(zeros_hbm_.at[pl.ds(r0, ROWS_PER_TILE)],
                        deg_sh.at[pl.ds(r0, ROWS_PER_TILE)])
        pltpu.sync_copy(ones_hbm_, ones_v)
        plsc.subcore_barrier()

        nb = jnp.where(s < 1, DB_BASE + 1, DB_BASE)
        dbase = c * DB_TOTAL + s * DB_BASE + jnp.minimum(s, 1)

        def dload(b, buf, sem):
            pltpu.async_copy(dst_hbm.at[pl.ds((dbase + b) * EB, EB)],
                             dst2_v.at[buf], sem)

        def dwait(buf, sem):
            pltpu.make_async_copy(dst_hbm.at[pl.ds(0, EB)], dst2_v.at[buf],
                                  sem).wait()

        dload(0, 0, dem0)
        dload(1, 1, dem1)

        def half(j, b, buf, sem):
            @pl.when(b < nb)
            def _():
                dwait(buf, sem)
                pltpu.sync_copy(ones_v, deg_sh.at[dst2_v.at[buf]], add=True)

                @pl.when(b + 2 < nb)
                def _():
                    dload(b + 2, buf, sem)

        def body(j, carry):
            half(j, 2 * j, 0, dem0)
            half(j, 2 * j + 1, 1, dem1)
            return carry

        lax.fori_loop(0, (DB_BASE + 2) // 2, body, 0)

        plsc.subcore_barrier()
        pltpu.sync_copy(deg_sh.at[pl.ds(r0, ROWS_PER_TILE)],
                        deg_out.at[pl.ds(c * N_PAD + r0, ROWS_PER_TILE)])

    return k(dst, zeros_hbm, ones_hbm)


def _layer_body(agg_lo, agg_hi, dega, degb, h, wlT, wrT, bl, g, b, out_ref):
    deg = dega[:, :1] + degb[:, :1]
    inv = 1.0 / jnp.maximum(deg, 1.0)
    mean = jnp.concatenate([agg_lo[...], agg_hi[...]], axis=1) * inv
    hb = h[...]
    out = (jnp.dot(mean, wlT[...], preferred_element_type=jnp.float32)
           + jnp.dot(hb, wrT[...], preferred_element_type=jnp.float32)
           + bl[...])
    nrm = jnp.maximum(jnp.sqrt(jnp.sum(out * out, axis=1, keepdims=True)), 1e-12)
    out = out / nrm
    mu = jnp.mean(out, axis=1, keepdims=True)
    var = jnp.mean((out - mu) ** 2, axis=1, keepdims=True)
    out = (out - mu) / jnp.sqrt(var + 1e-5) * g[...] + b[...]
    out = jnp.maximum(out, 0.0)
    out_ref[...] = out + hb


def _tc_layer(agg2, deg2, h, wlT, wrT, bl, g, b):
    full = lambda shape: pl.BlockSpec(shape, lambda i: (0,) * len(shape))
    nblk = N_PAD // BN
    return pl.pallas_call(
        _layer_body,
        grid=(GRID_N,),
        in_specs=[
            pl.BlockSpec((BN, HALF), lambda i: (i, 0)),
            pl.BlockSpec((BN, HALF), lambda i: (nblk + i, 0)),
            pl.BlockSpec((BN, HALF), lambda i: (i, 0)),
            pl.BlockSpec((BN, HALF), lambda i: (nblk + i, 0)),
            pl.BlockSpec((BN, HID), lambda i: (i, 0)),
            full((HID, HID)),
            full((HID, HID)),
            full((1, HID)),
            full((1, HID)),
            full((1, HID)),
        ],
        out_specs=pl.BlockSpec((BN, HID), lambda i: (i, 0)),
        out_shape=jax.ShapeDtypeStruct((N, HID), jnp.float32),
    )(agg2, agg2, deg2, deg2, h, wlT, wrT, bl, g, b)


def _layer_scores_body(agg_lo, agg_hi, dega, degb, h, wlT, wrT, bl, g, b,
                       w1T, b1, w2bd, b2, out_ref, s_ref, m_ref, m_scr):
    i = pl.program_id(0)
    deg = dega[:, :1] + degb[:, :1]
    inv = 1.0 / jnp.maximum(deg, 1.0)
    mean = jnp.concatenate([agg_lo[...], agg_hi[...]], axis=1) * inv
    hb = h[...]
    out = (jnp.dot(mean, wlT[...], preferred_element_type=jnp.float32)
           + jnp.dot(hb, wrT[...], preferred_element_type=jnp.float32)
           + bl[...])
    nrm = jnp.maximum(jnp.sqrt(jnp.sum(out * out, axis=1, keepdims=True)), 1e-12)
    out = out / nrm
    mu = jnp.mean(out, axis=1, keepdims=True)
    var = jnp.mean((out - mu) ** 2, axis=1, keepdims=True)
    out = (out - mu) / jnp.sqrt(var + 1e-5) * g[...] + b[...]
    out = jnp.maximum(out, 0.0) + hb
    out_ref[...] = out

    hid = jnp.tanh(jnp.dot(out, w1T[...], preferred_element_type=jnp.float32)
                   + b1[...])
    sv = jnp.dot(hid, w2bd[...], preferred_element_type=jnp.float32) + b2[...]
    s_ref[...] = sv
    rows = i * BN + lax.broadcasted_iota(jnp.int32, sv.shape, 0)
    sm = jnp.where(rows < N, sv, -jnp.inf)
    bm = jnp.max(sm, axis=0, keepdims=True)

    @pl.when(i == 0)
    def _():
        m_scr[:1, :ATT_HEADS] = bm

    @pl.when(i > 0)
    def _():
        m_scr[:1, :ATT_HEADS] = jnp.maximum(m_scr[:1, :ATT_HEADS], bm)

    @pl.when(i == GRID_N - 1)
    def _():
        m_ref[...] = m_scr[:1, :ATT_HEADS]


def _tc_layer_scores(agg2, deg2, h, wlT, wrT, bl, g, b, w1T, b1, w2bd, b2):
    full = lambda shape: pl.BlockSpec(shape, lambda i: (0,) * len(shape))
    nblk = N_PAD // BN
    return pl.pallas_call(
        _layer_scores_body,
        grid=(GRID_N,),
        in_specs=[
            pl.BlockSpec((BN, HALF), lambda i: (i, 0)),
            pl.BlockSpec((BN, HALF), lambda i: (nblk + i, 0)),
            pl.BlockSpec((BN, HALF), lambda i: (i, 0)),
            pl.BlockSpec((BN, HALF), lambda i: (nblk + i, 0)),
            pl.BlockSpec((BN, HID), lambda i: (i, 0)),
            full((HID, HID)),
            full((HID, HID)),
            full((1, HID)),
            full((1, HID)),
            full((1, HID)),
            full((HID, ATT_HEADS * ATT_DIM)),
            full((1, ATT_HEADS * ATT_DIM)),
            full((ATT_HEADS * ATT_DIM, ATT_HEADS)),
            full((1, ATT_HEADS)),
        ],
        out_specs=[
            pl.BlockSpec((BN, HID), lambda i: (i, 0)),
            pl.BlockSpec((BN, ATT_HEADS), lambda i: (i, 0)),
            pl.BlockSpec((1, ATT_HEADS), lambda i: (0, 0)),
        ],
        out_shape=[
            jax.ShapeDtypeStruct((N, HID), jnp.float32),
            jax.ShapeDtypeStruct((N, ATT_HEADS), jnp.float32),
            jax.ShapeDtypeStruct((1, ATT_HEADS), jnp.float32),
        ],
        scratch_shapes=[pltpu.VMEM((8, 128), jnp.float32)],
    )(agg2, agg2, deg2, deg2, h, wlT, wrT, bl, g, b, w1T, b1, w2bd, b2)


def _scores_body(h, w1T, b1, w2bd, b2, s_ref, m_ref, m_scr):
    i = pl.program_id(0)
    hid = jnp.tanh(jnp.dot(h[...], w1T[...], preferred_element_type=jnp.float32)
                   + b1[...])
    s = jnp.dot(hid, w2bd[...], preferred_element_type=jnp.float32) + b2[...]
    s_ref[...] = s
    rows = i * BN + lax.broadcasted_iota(jnp.int32, s.shape, 0)
    sm = jnp.where(rows < N, s, -jnp.inf)
    bm = jnp.max(sm, axis=0, keepdims=True)

    @pl.when(i == 0)
    def _():
        m_scr[:1, :ATT_HEADS] = bm

    @pl.when(i > 0)
    def _():
        m_scr[:1, :ATT_HEADS] = jnp.maximum(m_scr[:1, :ATT_HEADS], bm)

    @pl.when(i == GRID_N - 1)
    def _():
        m_ref[...] = m_scr[:1, :ATT_HEADS]


def _tc_scores(h, w1T, b1, w2bd, b2):
    full = lambda shape: pl.BlockSpec(shape, lambda i: (0,) * len(shape))
    return pl.pallas_call(
        _scores_body,
        grid=(GRID_N,),
        in_specs=[
            pl.BlockSpec((BN, HID), lambda i: (i, 0)),
            full((HID, ATT_HEADS * ATT_DIM)),
            full((1, ATT_HEADS * ATT_DIM)),
            full((ATT_HEADS * ATT_DIM, ATT_HEADS)),
            full((1, ATT_HEADS)),
        ],
        out_specs=[
            pl.BlockSpec((BN, ATT_HEADS), lambda i: (i, 0)),
            pl.BlockSpec((1, ATT_HEADS), lambda i: (0, 0)),
        ],
        out_shape=[
            jax.ShapeDtypeStruct((N, ATT_HEADS), jnp.float32),
            jax.ShapeDtypeStruct((1, ATT_HEADS), jnp.float32),
        ],
        scratch_shapes=[pltpu.VMEM((8, 128), jnp.float32)],
    )(h, w1T, b1, w2bd, b2)


def _pool_body(s, m, h, e_ref, z_ref, p_ref, z_scr, p_scr):
    i = pl.program_id(0)
    e = jnp.exp(s[...] - m[...])
    rows = i * BN + lax.broadcasted_iota(jnp.int32, e.shape, 0)
    e = jnp.where(rows < N, e, 0.0)
    e_ref[...] = e
    zb = jnp.sum(e, axis=0, keepdims=True)
    hb = h[...]
    hrows = i * BN + lax.broadcasted_iota(jnp.int32, hb.shape, 0)
    hb = jnp.where(hrows < N, hb, 0.0)
    pb = lax.dot_general(e, hb, (((0,), (0,)), ((), ())),
                         preferred_element_type=jnp.float32)

    @pl.when(i == 0)
    def _():
        z_scr[:1, :ATT_HEADS] = zb
        p_scr[...] = pb

    @pl.when(i > 0)
    def _():
        z_scr[:1, :ATT_HEADS] = z_scr[:1, :ATT_HEADS] + zb
        p_scr[...] = p_scr[...] + pb

    @pl.when(i == GRID_N - 1)
    def _():
        z_ref[...] = z_scr[:1, :ATT_HEADS]
        p_ref[...] = p_scr[...]


def _tc_pool(s, m, h):
    full = lambda shape: pl.BlockSpec(shape, lambda i: (0,) * len(shape))
    return pl.pallas_call(
        _pool_body,
        grid=(GRID_N,),
        in_specs=[
            pl.BlockSpec((BN, ATT_HEADS), lambda i: (i, 0)),
            full((1, ATT_HEADS)),
            pl.BlockSpec((BN, HID), lambda i: (i, 0)),
        ],
        out_specs=[
            pl.BlockSpec((BN, ATT_HEADS), lambda i: (i, 0)),
            pl.BlockSpec((1, ATT_HEADS), lambda i: (0, 0)),
            pl.BlockSpec((ATT_HEADS, HID), lambda i: (0, 0)),
        ],
        out_shape=[
            jax.ShapeDtypeStruct((N, ATT_HEADS), jnp.float32),
            jax.ShapeDtypeStruct((1, ATT_HEADS), jnp.float32),
            jax.ShapeDtypeStruct((ATT_HEADS, HID), jnp.float32),
        ],
        scratch_shapes=[pltpu.VMEM((8, 128), jnp.float32),
                        pltpu.VMEM((ATT_HEADS, HID), jnp.float32)],
    )(s, m, h)


def _ln_row(t, g, b):
    mu = jnp.mean(t, axis=1, keepdims=True)
    var = jnp.mean((t - mu) ** 2, axis=1, keepdims=True)
    return (t - mu) / jnp.sqrt(var + 1e-5) * g + b


def _final_body(e, z, p, wc1T, bc1, g1, b1n, wc2T, bc2, g2, b2n, wc3T, bc3,
                a_ref, probs_ref):
    i = pl.program_id(0)
    a_ref[...] = e[...] * (1.0 / z[...])

    @pl.when(i == GRID_N - 1)
    def _():
        invz = (1.0 / z[...]).reshape(ATT_HEADS, 1)
        zagg = jnp.mean(p[...] * invz, axis=0, keepdims=True)
        t = jnp.dot(zagg, wc1T[...], preferred_element_type=jnp.float32) + bc1[...]
        t = jnp.maximum(_ln_row(t, g1[...], b1n[...]), 0.0)
        t = jnp.dot(t, wc2T[...], preferred_element_type=jnp.float32) + bc2[...]
        t = jnp.maximum(_ln_row(t, g2[...], b2n[...]), 0.0)
        logits = jnp.dot(t, wc3T[...], preferred_element_type=jnp.float32) + bc3[...]
        mx = jnp.max(logits, axis=1, keepdims=True)
        ex = jnp.exp(logits - mx)
        probs_ref[...] = ex / jnp.sum(ex, axis=1, keepdims=True)


def _tc_final(e, z, p, cw):
    full = lambda shape: pl.BlockSpec(shape, lambda i: (0,) * len(shape))
    return pl.pallas_call(
        _final_body,
        grid=(GRID_N,),
        in_specs=[
            pl.BlockSpec((BN, ATT_HEADS), lambda i: (i, 0)),
            full((1, ATT_HEADS)),
            full((ATT_HEADS, HID)),
            full((HID, CLS)), full((1, CLS)), full((1, CLS)), full((1, CLS)),
            full((CLS, CLS // 2)), full((1, CLS // 2)), full((1, CLS // 2)),
            full((1, CLS // 2)),
            full((CLS // 2, NUM_CLASSES)), full((1, NUM_CLASSES)),
        ],
        out_specs=[
            pl.BlockSpec((BN, ATT_HEADS), lambda i: (i, 0)),
            pl.BlockSpec((1, NUM_CLASSES), lambda i: (0, 0)),
        ],
        out_shape=[
            jax.ShapeDtypeStruct((N, ATT_HEADS), jnp.float32),
            jax.ShapeDtypeStruct((1, NUM_CLASSES), jnp.float32),
        ],
    )(e, z, p, *cw)


def kernel(x, edge_index, params):
    src = edge_index[0]
    dst = edge_index[1]
    srcg = jnp.concatenate([src * 2, src * 2 + 1])
    zeros_hbm = jnp.zeros((N_PAD, HALF), jnp.float32)
    ones_hbm = jnp.ones((EB, HALF), jnp.float32)

    deg2 = _sc_degree(dst, zeros_hbm, ones_hbm)

    att = params['att']
    w1T = jnp.concatenate([ap['W1'] for ap in att], axis=0).T
    b1 = jnp.concatenate([ap['b1'] for ap in att]).reshape(1, -1)
    w2bd = jnp.zeros((ATT_HEADS * ATT_DIM, ATT_HEADS), jnp.float32)
    for hh, ap in enumerate(att):
        w2bd = w2bd.at[hh * ATT_DIM:(hh + 1) * ATT_DIM, hh].set(ap['W2'][0])
    b2 = jnp.stack([ap['b2'][0] for ap in att]).reshape(1, ATT_HEADS)

    lp0, lp1 = params['layers']
    agg2 = _sc_segment_sum(x.reshape(2 * N, HALF), srcg, dst, zeros_hbm)
    h = _tc_layer(agg2, deg2, x, lp0['W_l'].T, lp0['W_r'].T,
                  lp0['b_l'].reshape(1, HID), lp0['ln_g'].reshape(1, HID),
                  lp0['ln_b'].reshape(1, HID))
    agg2 = _sc_segment_sum(h.reshape(2 * N, HALF), srcg, dst, zeros_hbm)
    h, s, m = _tc_layer_scores(agg2, deg2, h, lp1['W_l'].T, lp1['W_r'].T,
                               lp1['b_l'].reshape(1, HID),
                               lp1['ln_g'].reshape(1, HID),
                               lp1['ln_b'].reshape(1, HID),
                               w1T, b1, w2bd, b2)
    e, z, p = _tc_pool(s, m, h)

    c = params['cls']
    cw = (c['W1'].T, c['b1'].reshape(1, -1), c['g1'].reshape(1, -1),
          c['b1n'].reshape(1, -1),
          c['W2'].T, c['b2'].reshape(1, -1), c['g2'].reshape(1, -1),
          c['b2n'].reshape(1, -1),
          c['W3'].T, c['b3'].reshape(1, -1))
    a, probs = _tc_final(e, z, p, cw)
    return probs[0], a


# speedup vs baseline: 1.2226x; 1.0004x over previous
import functools

import jax
import jax.numpy as jnp
from jax import lax
from jax.experimental import pallas as pl
from jax.experimental.pallas import tpu as pltpu
from jax.experimental.pallas import tpu_sc as plsc

N = 10000
E = 160000
D = 256
HID = 256
ATT_DIM = 128
ATT_HEADS = 4
CLS = 128
NUM_CLASSES = 7

HALF = D // 2
EB = 128
NB_TOTAL = E // EB
NSUB = 16
TB_MAX = 80
TB_LAST = NB_TOTAL - TB_MAX * (NSUB - 1)
TB_LAST_PAD = 56
NB_PADDED = TB_MAX * (NSUB - 1) + TB_LAST_PAD
N_PAD = 10240
ROWS_PER_TILE = N_PAD // NSUB

EHALF = E // 2
DB_TOTAL = EHALF // EB
DB_BASE = DB_TOTAL // NSUB
DB_REM = DB_TOTAL - DB_BASE * NSUB

BN = 1024
GRID_N = (N + BN - 1) // BN


def _sc_segment_sum(h2, srcg, dst, zeros_hbm, ones_hbm=None):
    with_deg = ones_hbm is not None
    mesh = plsc.VectorSubcoreMesh(core_axis_name="c", subcore_axis_name="s")

    @functools.partial(
        pl.kernel,
        out_type=([jax.ShapeDtypeStruct((2 * N_PAD, HALF), jnp.float32)] * 2
                  if with_deg else
                  jax.ShapeDtypeStruct((2 * N_PAD, HALF), jnp.float32)),
        mesh=mesh,
        scratch_types=[
            pltpu.VMEM_SHARED((N_PAD, HALF), jnp.float32),
            pltpu.VMEM((TB_MAX * EB,), jnp.int32),
            pltpu.VMEM((2, EB), jnp.int32),
            pltpu.VMEM((2, EB, HALF), jnp.float32),
            pltpu.SemaphoreType.DMA,
            pltpu.SemaphoreType.DMA,
            pltpu.SemaphoreType.DMA,
            pltpu.SemaphoreType.DMA,
        ],
    )
    def k(*args):
        if with_deg:
            (h2_hbm, srcg_hbm, dst_hbm, zeros_hbm_, ones_hbm_, agg_out,
             deg_out, agg_sh, idx_all, dst2_v, rows2,
             sem0, sem1, dem0, dem1) = args
        else:
            (h2_hbm, srcg_hbm, dst_hbm, zeros_hbm_, agg_out,
             agg_sh, idx_all, dst2_v, rows2, sem0, sem1, dem0, dem1) = args
        c = lax.axis_index("c")
        s = lax.axis_index("s")
        r0 = s * ROWS_PER_TILE
        pltpu.sync_copy(zeros_hbm_.at[pl.ds(r0, ROWS_PER_TILE)],
                        agg_sh.at[pl.ds(r0, ROWS_PER_TILE)])

        b0 = s * TB_MAX

        @pl.when(s < NSUB - 1)
        def _():
            pltpu.sync_copy(srcg_hbm.at[pl.ds(c * E + b0 * EB, TB_MAX * EB)],
                            idx_all)

        @pl.when(s == NSUB - 1)
        def _():
            pltpu.sync_copy(srcg_hbm.at[pl.ds(c * E + b0 * EB, TB_LAST * EB)],
                            idx_all.at[pl.ds(0, TB_LAST * EB)])

        plsc.subcore_barrier()

        nh = jnp.where(s < NSUB - 1, TB_MAX // 2, TB_LAST // 2)

        def gather(b, buf, sem):
            pltpu.async_copy(h2_hbm.at[idx_all.at[pl.ds(b * EB, EB)]],
                             rows2.at[buf], sem)

        def gwait(buf, sem):
            pltpu.make_async_copy(h2_hbm.at[pl.ds(0, EB)], rows2.at[buf], sem).wait()

        def dload(b, buf, sem):
            pltpu.async_copy(dst_hbm.at[pl.ds((b0 + b) * EB, EB)],
                             dst2_v.at[buf], sem)

        def dwait(buf, sem):
            pltpu.make_async_copy(dst_hbm.at[pl.ds(0, EB)], dst2_v.at[buf],
                                  sem).wait()

        gather(0, 0, sem0)
        gather(1, 1, sem1)
        dload(0, 0, dem0)
        dload(1, 1, dem1)

        def body(j, carry):
            b_even = 2 * j

            gwait(0, sem0)
            dwait(0, dem0)
            pltpu.sync_copy(rows2.at[0], agg_sh.at[dst2_v.at[0]], add=True)

            @pl.when(j < nh - 1)
            def _():
                gather(b_even + 2, 0, sem0)
                dload(b_even + 2, 0, dem0)

            gwait(1, sem1)
            dwait(1, dem1)
            pltpu.sync_copy(rows2.at[1], agg_sh.at[dst2_v.at[1]], add=True)

            @pl.when(j < nh - 1)
            def _():
                gather(b_even + 3, 1, sem1)
                dload(b_even + 3, 1, dem1)

            return carry

        lax.fori_loop(0, nh, body, 0)

        plsc.subcore_barrier()
        pltpu.sync_copy(agg_sh.at[pl.ds(r0, ROWS_PER_TILE)],
                        agg_out.at[pl.ds(c * N_PAD + r0, ROWS_PER_TILE)])

        if with_deg:
            plsc.subcore_barrier()
            pltpu.sync_copy(zeros_hbm_.at[pl.ds(r0, ROWS_PER_TILE)],
                            agg_sh.at[pl.ds(r0, ROWS_PER_TILE)])
            pltpu.sync_copy(ones_hbm_, rows2.at[0])
            plsc.subcore_barrier()

            dnb = jnp.where(s < 1, DB_BASE + 1, DB_BASE)
            dbase = c * DB_TOTAL + s * DB_BASE + jnp.minimum(s, 1)

            def ddload(b, buf, sem):
                pltpu.async_copy(dst_hbm.at[pl.ds((dbase + b) * EB, EB)],
                                 dst2_v.at[buf], sem)

            def ddwait(buf, sem):
                pltpu.make_async_copy(dst_hbm.at[pl.ds(0, EB)], dst2_v.at[buf],
                                      sem).wait()

            ddload(0, 0, dem0)
            ddload(1, 1, dem1)

            def dhalf(b, buf, sem):
                @pl.when(b < dnb)
                def _():
                    ddwait(buf, sem)
                    pltpu.sync_copy(rows2.at[0], agg_sh.at[dst2_v.at[buf]],
                                    add=True)

                    @pl.when(b + 2 < dnb)
                    def _():
                        ddload(b + 2, buf, sem)

            def dbody(j, carry):
                dhalf(2 * j, 0, dem0)
                dhalf(2 * j + 1, 1, dem1)
                return carry

            lax.fori_loop(0, (DB_BASE + 2) // 2, dbody, 0)

            plsc.subcore_barrier()
            pltpu.sync_copy(agg_sh.at[pl.ds(r0, ROWS_PER_TILE)],
                            deg_out.at[pl.ds(c * N_PAD + r0, ROWS_PER_TILE)])

    if with_deg:
        return k(h2, srcg, dst, zeros_hbm, ones_hbm)
    return k(h2, srcg, dst, zeros_hbm)


def _sc_degree(dst, zeros_hbm, ones_hbm):
    mesh = plsc.VectorSubcoreMesh(core_axis_name="c", subcore_axis_name="s")

    @functools.partial(
        pl.kernel,
        out_type=jax.ShapeDtypeStruct((2 * N_PAD, HALF), jnp.float32),
        mesh=mesh,
        scratch_types=[
            pltpu.VMEM_SHARED((N_PAD, HALF), jnp.float32),
            pltpu.VMEM((2, EB), jnp.int32),
            pltpu.VMEM((EB, HALF), jnp.float32),
            pltpu.SemaphoreType.DMA,
            pltpu.SemaphoreType.DMA,
        ],
    )
    def k(dst_hbm, zeros_hbm_, ones_hbm_, deg_out, deg_sh, dst2_v, ones_v,
          dem0, dem1):
        c = lax.axis_index("c")
        s = lax.axis_index("s")
        r0 = s * ROWS_PER_TILE
        pltpu.sync_copy(zeros_hbm_.at[pl.ds(r0, ROWS_PER_TILE)],
                        deg_sh.at[pl.ds(r0, ROWS_PER_TILE)])
        pltpu.sync_copy(ones_hbm_, ones_v)
        plsc.subcore_barrier()

        nb = jnp.where(s < 1, DB_BASE + 1, DB_BASE)
        dbase = c * DB_TOTAL + s * DB_BASE + jnp.minimum(s, 1)

        def dload(b, buf, sem):
            pltpu.async_copy(dst_hbm.at[pl.ds((dbase + b) * EB, EB)],
                             dst2_v.at[buf], sem)

        def dwait(buf, sem):
            pltpu.make_async_copy(dst_hbm.at[pl.ds(0, EB)], dst2_v.at[buf],
                                  sem).wait()

        dload(0, 0, dem0)
        dload(1, 1, dem1)

        def half(j, b, buf, sem):
            @pl.when(b < nb)
            def _():
                dwait(buf, sem)
                pltpu.sync_copy(ones_v, deg_sh.at[dst2_v.at[buf]], add=True)

                @pl.when(b + 2 < nb)
                def _():
                    dload(b + 2, buf, sem)

        def body(j, carry):
            half(j, 2 * j, 0, dem0)
            half(j, 2 * j + 1, 1, dem1)
            return carry

        lax.fori_loop(0, (DB_BASE + 2) // 2, body, 0)

        plsc.subcore_barrier()
        pltpu.sync_copy(deg_sh.at[pl.ds(r0, ROWS_PER_TILE)],
                        deg_out.at[pl.ds(c * N_PAD + r0, ROWS_PER_TILE)])

    return k(dst, zeros_hbm, ones_hbm)


def _layer_body(agg_lo, agg_hi, dega, degb, h, wlT, wrT, bl, g, b, out_ref):
    deg = dega[:, :1] + degb[:, :1]
    inv = 1.0 / jnp.maximum(deg, 1.0)
    mean = jnp.concatenate([agg_lo[...], agg_hi[...]], axis=1) * inv
    hb = h[...]
    out = (jnp.dot(mean, wlT[...], preferred_element_type=jnp.float32)
           + jnp.dot(hb, wrT[...], preferred_element_type=jnp.float32)
           + bl[...])
    nrm = jnp.maximum(jnp.sqrt(jnp.sum(out * out, axis=1, keepdims=True)), 1e-12)
    out = out / nrm
    mu = jnp.mean(out, axis=1, keepdims=True)
    var = jnp.mean((out - mu) ** 2, axis=1, keepdims=True)
    out = (out - mu) / jnp.sqrt(var + 1e-5) * g[...] + b[...]
    out = jnp.maximum(out, 0.0)
    out_ref[...] = out + hb


def _tc_layer(agg2, deg2, h, wlT, wrT, bl, g, b):
    full = lambda shape: pl.BlockSpec(shape, lambda i: (0,) * len(shape))
    nblk = N_PAD // BN
    return pl.pallas_call(
        _layer_body,
        grid=(GRID_N,),
        in_specs=[
            pl.BlockSpec((BN, HALF), lambda i: (i, 0)),
            pl.BlockSpec((BN, HALF), lambda i: (nblk + i, 0)),
            pl.BlockSpec((BN, HALF), lambda i: (i, 0)),
            pl.BlockSpec((BN, HALF), lambda i: (nblk + i, 0)),
            pl.BlockSpec((BN, HID), lambda i: (i, 0)),
            full((HID, HID)),
            full((HID, HID)),
            full((1, HID)),
            full((1, HID)),
            full((1, HID)),
        ],
        out_specs=pl.BlockSpec((BN, HID), lambda i: (i, 0)),
        out_shape=jax.ShapeDtypeStruct((N, HID), jnp.float32),
    )(agg2, agg2, deg2, deg2, h, wlT, wrT, bl, g, b)


def _layer_scores_body(agg_lo, agg_hi, dega, degb, h, wlT, wrT, bl, g, b,
                       w1T, b1, w2bd, b2, out_ref, s_ref, m_ref, m_scr):
    i = pl.program_id(0)
    deg = dega[:, :1] + degb[:, :1]
    inv = 1.0 / jnp.maximum(deg, 1.0)
    mean = jnp.concatenate([agg_lo[...], agg_hi[...]], axis=1) * inv
    hb = h[...]
    out = (jnp.dot(mean, wlT[...], preferred_element_type=jnp.float32)
           + jnp.dot(hb, wrT[...], preferred_element_type=jnp.float32)
           + bl[...])
    nrm = jnp.maximum(jnp.sqrt(jnp.sum(out * out, axis=1, keepdims=True)), 1e-12)
    out = out / nrm
    mu = jnp.mean(out, axis=1, keepdims=True)
    var = jnp.mean((out - mu) ** 2, axis=1, keepdims=True)
    out = (out - mu) / jnp.sqrt(var + 1e-5) * g[...] + b[...]
    out = jnp.maximum(out, 0.0) + hb
    out_ref[...] = out

    hid = jnp.tanh(jnp.dot(out, w1T[...], preferred_element_type=jnp.float32)
                   + b1[...])
    sv = jnp.dot(hid, w2bd[...], preferred_element_type=jnp.float32) + b2[...]
    s_ref[...] = sv
    rows = i * BN + lax.broadcasted_iota(jnp.int32, sv.shape, 0)
    sm = jnp.where(rows < N, sv, -jnp.inf)
    bm = jnp.max(sm, axis=0, keepdims=True)

    @pl.when(i == 0)
    def _():
        m_scr[:1, :ATT_HEADS] = bm

    @pl.when(i > 0)
    def _():
        m_scr[:1, :ATT_HEADS] = jnp.maximum(m_scr[:1, :ATT_HEADS], bm)

    @pl.when(i == GRID_N - 1)
    def _():
        m_ref[...] = m_scr[:1, :ATT_HEADS]


def _tc_layer_scores(agg2, deg2, h, wlT, wrT, bl, g, b, w1T, b1, w2bd, b2):
    full = lambda shape: pl.BlockSpec(shape, lambda i: (0,) * len(shape))
    nblk = N_PAD // BN
    return pl.pallas_call(
        _layer_scores_body,
        grid=(GRID_N,),
        in_specs=[
            pl.BlockSpec((BN, HALF), lambda i: (i, 0)),
            pl.BlockSpec((BN, HALF), lambda i: (nblk + i, 0)),
            pl.BlockSpec((BN, HALF), lambda i: (i, 0)),
            pl.BlockSpec((BN, HALF), lambda i: (nblk + i, 0)),
            pl.BlockSpec((BN, HID), lambda i: (i, 0)),
            full((HID, HID)),
            full((HID, HID)),
            full((1, HID)),
            full((1, HID)),
            full((1, HID)),
            full((HID, ATT_HEADS * ATT_DIM)),
            full((1, ATT_HEADS * ATT_DIM)),
            full((ATT_HEADS * ATT_DIM, ATT_HEADS)),
            full((1, ATT_HEADS)),
        ],
        out_specs=[
            pl.BlockSpec((BN, HID), lambda i: (i, 0)),
            pl.BlockSpec((BN, ATT_HEADS), lambda i: (i, 0)),
            pl.BlockSpec((1, ATT_HEADS), lambda i: (0, 0)),
        ],
        out_shape=[
            jax.ShapeDtypeStruct((N, HID), jnp.float32),
            jax.ShapeDtypeStruct((N, ATT_HEADS), jnp.float32),
            jax.ShapeDtypeStruct((1, ATT_HEADS), jnp.float32),
        ],
        scratch_shapes=[pltpu.VMEM((8, 128), jnp.float32)],
    )(agg2, agg2, deg2, deg2, h, wlT, wrT, bl, g, b, w1T, b1, w2bd, b2)


def _scores_body(h, w1T, b1, w2bd, b2, s_ref, m_ref, m_scr):
    i = pl.program_id(0)
    hid = jnp.tanh(jnp.dot(h[...], w1T[...], preferred_element_type=jnp.float32)
                   + b1[...])
    s = jnp.dot(hid, w2bd[...], preferred_element_type=jnp.float32) + b2[...]
    s_ref[...] = s
    rows = i * BN + lax.broadcasted_iota(jnp.int32, s.shape, 0)
    sm = jnp.where(rows < N, s, -jnp.inf)
    bm = jnp.max(sm, axis=0, keepdims=True)

    @pl.when(i == 0)
    def _():
        m_scr[:1, :ATT_HEADS] = bm

    @pl.when(i > 0)
    def _():
        m_scr[:1, :ATT_HEADS] = jnp.maximum(m_scr[:1, :ATT_HEADS], bm)

    @pl.when(i == GRID_N - 1)
    def _():
        m_ref[...] = m_scr[:1, :ATT_HEADS]


def _tc_scores(h, w1T, b1, w2bd, b2):
    full = lambda shape: pl.BlockSpec(shape, lambda i: (0,) * len(shape))
    return pl.pallas_call(
        _scores_body,
        grid=(GRID_N,),
        in_specs=[
            pl.BlockSpec((BN, HID), lambda i: (i, 0)),
            full((HID, ATT_HEADS * ATT_DIM)),
            full((1, ATT_HEADS * ATT_DIM)),
            full((ATT_HEADS * ATT_DIM, ATT_HEADS)),
            full((1, ATT_HEADS)),
        ],
        out_specs=[
            pl.BlockSpec((BN, ATT_HEADS), lambda i: (i, 0)),
            pl.BlockSpec((1, ATT_HEADS), lambda i: (0, 0)),
        ],
        out_shape=[
            jax.ShapeDtypeStruct((N, ATT_HEADS), jnp.float32),
            jax.ShapeDtypeStruct((1, ATT_HEADS), jnp.float32),
        ],
        scratch_shapes=[pltpu.VMEM((8, 128), jnp.float32)],
    )(h, w1T, b1, w2bd, b2)


def _pool_body(s, m, h, e_ref, z_ref, p_ref, z_scr, p_scr):
    i = pl.program_id(0)
    e = jnp.exp(s[...] - m[...])
    rows = i * BN + lax.broadcasted_iota(jnp.int32, e.shape, 0)
    e = jnp.where(rows < N, e, 0.0)
    e_ref[...] = e
    zb = jnp.sum(e, axis=0, keepdims=True)
    hb = h[...]
    hrows = i * BN + lax.broadcasted_iota(jnp.int32, hb.shape, 0)
    hb = jnp.where(hrows < N, hb, 0.0)
    pb = lax.dot_general(e, hb, (((0,), (0,)), ((), ())),
                         preferred_element_type=jnp.float32)

    @pl.when(i == 0)
    def _():
        z_scr[:1, :ATT_HEADS] = zb
        p_scr[...] = pb

    @pl.when(i > 0)
    def _():
        z_scr[:1, :ATT_HEADS] = z_scr[:1, :ATT_HEADS] + zb
        p_scr[...] = p_scr[...] + pb

    @pl.when(i == GRID_N - 1)
    def _():
        z_ref[...] = z_scr[:1, :ATT_HEADS]
        p_ref[...] = p_scr[...]


def _tc_pool(s, m, h):
    full = lambda shape: pl.BlockSpec(shape, lambda i: (0,) * len(shape))
    return pl.pallas_call(
        _pool_body,
        grid=(GRID_N,),
        in_specs=[
            pl.BlockSpec((BN, ATT_HEADS), lambda i: (i, 0)),
            full((1, ATT_HEADS)),
            pl.BlockSpec((BN, HID), lambda i: (i, 0)),
        ],
        out_specs=[
            pl.BlockSpec((BN, ATT_HEADS), lambda i: (i, 0)),
            pl.BlockSpec((1, ATT_HEADS), lambda i: (0, 0)),
            pl.BlockSpec((ATT_HEADS, HID), lambda i: (0, 0)),
        ],
        out_shape=[
            jax.ShapeDtypeStruct((N, ATT_HEADS), jnp.float32),
            jax.ShapeDtypeStruct((1, ATT_HEADS), jnp.float32),
            jax.ShapeDtypeStruct((ATT_HEADS, HID), jnp.float32),
        ],
        scratch_shapes=[pltpu.VMEM((8, 128), jnp.float32),
                        pltpu.VMEM((ATT_HEADS, HID), jnp.float32)],
    )(s, m, h)


def _ln_row(t, g, b):
    mu = jnp.mean(t, axis=1, keepdims=True)
    var = jnp.mean((t - mu) ** 2, axis=1, keepdims=True)
    return (t - mu) / jnp.sqrt(var + 1e-5) * g + b


def _final_body(e, z, p, wc1T, bc1, g1, b1n, wc2T, bc2, g2, b2n, wc3T, bc3,
                a_ref, probs_ref):
    i = pl.program_id(0)
    a_ref[...] = e[...] * (1.0 / z[...])

    @pl.when(i == GRID_N - 1)
    def _():
        invz = (1.0 / z[...]).reshape(ATT_HEADS, 1)
        zagg = jnp.mean(p[...] * invz, axis=0, keepdims=True)
        t = jnp.dot(zagg, wc1T[...], preferred_element_type=jnp.float32) + bc1[...]
        t = jnp.maximum(_ln_row(t, g1[...], b1n[...]), 0.0)
        t = jnp.dot(t, wc2T[...], preferred_element_type=jnp.float32) + bc2[...]
        t = jnp.maximum(_ln_row(t, g2[...], b2n[...]), 0.0)
        logits = jnp.dot(t, wc3T[...], preferred_element_type=jnp.float32) + bc3[...]
        mx = jnp.max(logits, axis=1, keepdims=True)
        ex = jnp.exp(logits - mx)
        probs_ref[...] = ex / jnp.sum(ex, axis=1, keepdims=True)


def _tc_final(e, z, p, cw):
    full = lambda shape: pl.BlockSpec(shape, lambda i: (0,) * len(shape))
    return pl.pallas_call(
        _final_body,
        grid=(GRID_N,),
        in_specs=[
            pl.BlockSpec((BN, ATT_HEADS), lambda i: (i, 0)),
            full((1, ATT_HEADS)),
            full((ATT_HEADS, HID)),
            full((HID, CLS)), full((1, CLS)), full((1, CLS)), full((1, CLS)),
            full((CLS, CLS // 2)), full((1, CLS // 2)), full((1, CLS // 2)),
            full((1, CLS // 2)),
            full((CLS // 2, NUM_CLASSES)), full((1, NUM_CLASSES)),
        ],
        out_specs=[
            pl.BlockSpec((BN, ATT_HEADS), lambda i: (i, 0)),
            pl.BlockSpec((1, NUM_CLASSES), lambda i: (0, 0)),
        ],
        out_shape=[
            jax.ShapeDtypeStruct((N, ATT_HEADS), jnp.float32),
            jax.ShapeDtypeStruct((1, NUM_CLASSES), jnp.float32),
        ],
    )(e, z, p, *cw)


def kernel(x, edge_index, params):
    src = edge_index[0]
    dst = edge_index[1]
    srcg = jnp.concatenate([src * 2, src * 2 + 1])
    zeros_hbm = jnp.zeros((N_PAD, HALF), jnp.float32)
    ones_hbm = jnp.ones((EB, HALF), jnp.float32)

    att = params['att']
    w1T = jnp.concatenate([ap['W1'] for ap in att], axis=0).T
    b1 = jnp.concatenate([ap['b1'] for ap in att]).reshape(1, -1)
    w2bd = jnp.zeros((ATT_HEADS * ATT_DIM, ATT_HEADS), jnp.float32)
    for hh, ap in enumerate(att):
        w2bd = w2bd.at[hh * ATT_DIM:(hh + 1) * ATT_DIM, hh].set(ap['W2'][0])
    b2 = jnp.stack([ap['b2'][0] for ap in att]).reshape(1, ATT_HEADS)

    lp0, lp1 = params['layers']
    agg2, deg2 = _sc_segment_sum(x.reshape(2 * N, HALF), srcg, dst, zeros_hbm,
                                 ones_hbm)
    h = _tc_layer(agg2, deg2, x, lp0['W_l'].T, lp0['W_r'].T,
                  lp0['b_l'].reshape(1, HID), lp0['ln_g'].reshape(1, HID),
                  lp0['ln_b'].reshape(1, HID))
    agg2 = _sc_segment_sum(h.reshape(2 * N, HALF), srcg, dst, zeros_hbm)
    h, s, m = _tc_layer_scores(agg2, deg2, h, lp1['W_l'].T, lp1['W_r'].T,
                               lp1['b_l'].reshape(1, HID),
                               lp1['ln_g'].reshape(1, HID),
                               lp1['ln_b'].reshape(1, HID),
                               w1T, b1, w2bd, b2)
    e, z, p = _tc_pool(s, m, h)

    c = params['cls']
    cw = (c['W1'].T, c['b1'].reshape(1, -1), c['g1'].reshape(1, -1),
          c['b1n'].reshape(1, -1),
          c['W2'].T, c['b2'].reshape(1, -1), c['g2'].reshape(1, -1),
          c['b2n'].reshape(1, -1),
          c['W3'].T, c['b3'].reshape(1, -1))
    a, probs = _tc_final(e, z, p, cw)
    return probs[0], a


# speedup vs baseline: 1.2608x; 1.0312x over previous
import functools

import jax
import jax.numpy as jnp
from jax import lax
from jax.experimental import pallas as pl
from jax.experimental.pallas import tpu as pltpu
from jax.experimental.pallas import tpu_sc as plsc

N = 10000
E = 160000
D = 256
HID = 256
ATT_DIM = 128
ATT_HEADS = 4
CLS = 128
NUM_CLASSES = 7

HALF = D // 2
EB = 128
NB_TOTAL = E // EB
NSUB = 16
TB_MAX = 80
TB_LAST = NB_TOTAL - TB_MAX * (NSUB - 1)
TB_LAST_PAD = 56
NB_PADDED = TB_MAX * (NSUB - 1) + TB_LAST_PAD
N_PAD = 10240
ROWS_PER_TILE = N_PAD // NSUB

EHALF = E // 2
DB_TOTAL = EHALF // EB
DB_BASE = DB_TOTAL // NSUB
DB_REM = DB_TOTAL - DB_BASE * NSUB

BN = 1024
GRID_N = (N + BN - 1) // BN


def _sc_segment_sum(h2, srcg, dst, zeros_hbm, ones_hbm=None):
    with_deg = ones_hbm is not None
    mesh = plsc.VectorSubcoreMesh(core_axis_name="c", subcore_axis_name="s")

    @functools.partial(
        pl.kernel,
        out_type=([jax.ShapeDtypeStruct((2 * N_PAD, HALF), jnp.float32)] * 2
                  if with_deg else
                  jax.ShapeDtypeStruct((2 * N_PAD, HALF), jnp.float32)),
        mesh=mesh,
        scratch_types=[
            pltpu.VMEM_SHARED((N_PAD, HALF), jnp.float32),
            pltpu.VMEM((TB_MAX * EB,), jnp.int32),
            pltpu.VMEM((2, EB), jnp.int32),
            pltpu.VMEM((2, EB, HALF), jnp.float32),
            pltpu.SemaphoreType.DMA,
            pltpu.SemaphoreType.DMA,
            pltpu.SemaphoreType.DMA,
            pltpu.SemaphoreType.DMA,
        ],
    )
    def k(*args):
        if with_deg:
            (h2_hbm, srcg_hbm, dst_hbm, zeros_hbm_, ones_hbm_, agg_out,
             deg_out, agg_sh, idx_all, dst2_v, rows2,
             sem0, sem1, dem0, dem1) = args
        else:
            (h2_hbm, srcg_hbm, dst_hbm, zeros_hbm_, agg_out,
             agg_sh, idx_all, dst2_v, rows2, sem0, sem1, dem0, dem1) = args
        c = lax.axis_index("c")
        s = lax.axis_index("s")
        r0 = s * ROWS_PER_TILE
        pltpu.sync_copy(zeros_hbm_.at[pl.ds(r0, ROWS_PER_TILE)],
                        agg_sh.at[pl.ds(r0, ROWS_PER_TILE)])

        b0 = s * TB_MAX

        @pl.when(s < NSUB - 1)
        def _():
            pltpu.sync_copy(srcg_hbm.at[pl.ds(c * E + b0 * EB, TB_MAX * EB)],
                            idx_all)

        @pl.when(s == NSUB - 1)
        def _():
            pltpu.sync_copy(srcg_hbm.at[pl.ds(c * E + b0 * EB, TB_LAST * EB)],
                            idx_all.at[pl.ds(0, TB_LAST * EB)])

        plsc.subcore_barrier()

        nh = jnp.where(s < NSUB - 1, TB_MAX // 2, TB_LAST // 2)

        def gather(b, buf, sem):
            pltpu.async_copy(h2_hbm.at[idx_all.at[pl.ds(b * EB, EB)]],
                             rows2.at[buf], sem)

        def gwait(buf, sem):
            pltpu.make_async_copy(h2_hbm.at[pl.ds(0, EB)], rows2.at[buf], sem).wait()

        def dload(b, buf, sem):
            pltpu.async_copy(dst_hbm.at[pl.ds((b0 + b) * EB, EB)],
                             dst2_v.at[buf], sem)

        def dwait(buf, sem):
            pltpu.make_async_copy(dst_hbm.at[pl.ds(0, EB)], dst2_v.at[buf],
                                  sem).wait()

        gather(0, 0, sem0)
        gather(1, 1, sem1)
        dload(0, 0, dem0)
        dload(1, 1, dem1)

        def body(j, carry):
            b_even = 2 * j

            gwait(0, sem0)
            dwait(0, dem0)
            pltpu.sync_copy(rows2.at[0], agg_sh.at[dst2_v.at[0]], add=True)

            @pl.when(j < nh - 1)
            def _():
                gather(b_even + 2, 0, sem0)
                dload(b_even + 2, 0, dem0)

            gwait(1, sem1)
            dwait(1, dem1)
            pltpu.sync_copy(rows2.at[1], agg_sh.at[dst2_v.at[1]], add=True)

            @pl.when(j < nh - 1)
            def _():
                gather(b_even + 3, 1, sem1)
                dload(b_even + 3, 1, dem1)

            return carry

        lax.fori_loop(0, nh, body, 0)

        plsc.subcore_barrier()
        pltpu.sync_copy(agg_sh.at[pl.ds(r0, ROWS_PER_TILE)],
                        agg_out.at[pl.ds(c * N_PAD + r0, ROWS_PER_TILE)])

        if with_deg:
            plsc.subcore_barrier()
            pltpu.sync_copy(zeros_hbm_.at[pl.ds(r0, ROWS_PER_TILE)],
                            agg_sh.at[pl.ds(r0, ROWS_PER_TILE)])
            pltpu.sync_copy(ones_hbm_, rows2.at[0])
            plsc.subcore_barrier()

            dnb = jnp.where(s < 1, DB_BASE + 1, DB_BASE)
            dbase = c * DB_TOTAL + s * DB_BASE + jnp.minimum(s, 1)

            def ddload(b, buf, sem):
                pltpu.async_copy(dst_hbm.at[pl.ds((dbase + b) * EB, EB)],
                                 dst2_v.at[buf], sem)

            def ddwait(buf, sem):
                pltpu.make_async_copy(dst_hbm.at[pl.ds(0, EB)], dst2_v.at[buf],
                                      sem).wait()

            ddload(0, 0, dem0)
            ddload(1, 1, dem1)

            def dhalf(b, buf, sem):
                @pl.when(b < dnb)
                def _():
                    ddwait(buf, sem)
                    pltpu.sync_copy(rows2.at[0], agg_sh.at[dst2_v.at[buf]],
                                    add=True)

                    @pl.when(b + 2 < dnb)
                    def _():
                        ddload(b + 2, buf, sem)

            def dbody(j, carry):
                dhalf(2 * j, 0, dem0)
                dhalf(2 * j + 1, 1, dem1)
                return carry

            lax.fori_loop(0, (DB_BASE + 2) // 2, dbody, 0)

            plsc.subcore_barrier()
            pltpu.sync_copy(agg_sh.at[pl.ds(r0, ROWS_PER_TILE)],
                            deg_out.at[pl.ds(c * N_PAD + r0, ROWS_PER_TILE)])

    if with_deg:
        return k(h2, srcg, dst, zeros_hbm, ones_hbm)
    return k(h2, srcg, dst, zeros_hbm)


def _sc_degree(dst, zeros_hbm, ones_hbm):
    mesh = plsc.VectorSubcoreMesh(core_axis_name="c", subcore_axis_name="s")

    @functools.partial(
        pl.kernel,
        out_type=jax.ShapeDtypeStruct((2 * N_PAD, HALF), jnp.float32),
        mesh=mesh,
        scratch_types=[
            pltpu.VMEM_SHARED((N_PAD, HALF), jnp.float32),
            pltpu.VMEM((2, EB), jnp.int32),
            pltpu.VMEM((EB, HALF), jnp.float32),
            pltpu.SemaphoreType.DMA,
            pltpu.SemaphoreType.DMA,
        ],
    )
    def k(dst_hbm, zeros_hbm_, ones_hbm_, deg_out, deg_sh, dst2_v, ones_v,
          dem0, dem1):
        c = lax.axis_index("c")
        s = lax.axis_index("s")
        r0 = s * ROWS_PER_TILE
        pltpu.sync_copy(zeros_hbm_.at[pl.ds(r0, ROWS_PER_TILE)],
                        deg_sh.at[pl.ds(r0, ROWS_PER_TILE)])
        pltpu.sync_copy(ones_hbm_, ones_v)
        plsc.subcore_barrier()

        nb = jnp.where(s < 1, DB_BASE + 1, DB_BASE)
        dbase = c * DB_TOTAL + s * DB_BASE + jnp.minimum(s, 1)

        def dload(b, buf, sem):
            pltpu.async_copy(dst_hbm.at[pl.ds((dbase + b) * EB, EB)],
                             dst2_v.at[buf], sem)

        def dwait(buf, sem):
            pltpu.make_async_copy(dst_hbm.at[pl.ds(0, EB)], dst2_v.at[buf],
                                  sem).wait()

        dload(0, 0, dem0)
        dload(1, 1, dem1)

        def half(j, b, buf, sem):
            @pl.when(b < nb)
            def _():
                dwait(buf, sem)
                pltpu.sync_copy(ones_v, deg_sh.at[dst2_v.at[buf]], add=True)

                @pl.when(b + 2 < nb)
                def _():
                    dload(b + 2, buf, sem)

        def body(j, carry):
            half(j, 2 * j, 0, dem0)
            half(j, 2 * j + 1, 1, dem1)
            return carry

        lax.fori_loop(0, (DB_BASE + 2) // 2, body, 0)

        plsc.subcore_barrier()
        pltpu.sync_copy(deg_sh.at[pl.ds(r0, ROWS_PER_TILE)],
                        deg_out.at[pl.ds(c * N_PAD + r0, ROWS_PER_TILE)])

    return k(dst, zeros_hbm, ones_hbm)


def _layer_body(agg_lo, agg_hi, dega, degb, h, wlT, wrT, bl, g, b, out_ref):
    deg = dega[:, :1] + degb[:, :1]
    inv = 1.0 / jnp.maximum(deg, 1.0)
    mean = jnp.concatenate([agg_lo[...], agg_hi[...]], axis=1) * inv
    hb = h[...]
    out = (jnp.dot(mean, wlT[...], preferred_element_type=jnp.float32)
           + jnp.dot(hb, wrT[...], preferred_element_type=jnp.float32)
           + bl[...])
    nrm = jnp.maximum(jnp.sqrt(jnp.sum(out * out, axis=1, keepdims=True)), 1e-12)
    out = out / nrm
    mu = jnp.mean(out, axis=1, keepdims=True)
    var = jnp.mean((out - mu) ** 2, axis=1, keepdims=True)
    out = (out - mu) / jnp.sqrt(var + 1e-5) * g[...] + b[...]
    out = jnp.maximum(out, 0.0)
    out_ref[...] = out + hb


def _tc_layer(agg2, deg2, h, wlT, wrT, bl, g, b):
    full = lambda shape: pl.BlockSpec(shape, lambda i: (0,) * len(shape))
    nblk = N_PAD // BN
    return pl.pallas_call(
        _layer_body,
        grid=(GRID_N,),
        in_specs=[
            pl.BlockSpec((BN, HALF), lambda i: (i, 0)),
            pl.BlockSpec((BN, HALF), lambda i: (nblk + i, 0)),
            pl.BlockSpec((BN, HALF), lambda i: (i, 0)),
            pl.BlockSpec((BN, HALF), lambda i: (nblk + i, 0)),
            pl.BlockSpec((BN, HID), lambda i: (i, 0)),
            full((HID, HID)),
            full((HID, HID)),
            full((1, HID)),
            full((1, HID)),
            full((1, HID)),
        ],
        out_specs=pl.BlockSpec((BN, HID), lambda i: (i, 0)),
        out_shape=jax.ShapeDtypeStruct((N, HID), jnp.float32),
    )(agg2, agg2, deg2, deg2, h, wlT, wrT, bl, g, b)


def _layer_scores_body(agg_lo, agg_hi, dega, degb, h, wlT, wrT, bl, g, b,
                       w1T, b1, w2bd, b2, out_ref, s_ref, m_ref, z_ref, p_ref,
                       m_scr, z_scr, p_scr):
    i = pl.program_id(0)
    deg = dega[:, :1] + degb[:, :1]
    inv = 1.0 / jnp.maximum(deg, 1.0)
    mean = jnp.concatenate([agg_lo[...], agg_hi[...]], axis=1) * inv
    hb = h[...]
    out = (jnp.dot(mean, wlT[...], preferred_element_type=jnp.float32)
           + jnp.dot(hb, wrT[...], preferred_element_type=jnp.float32)
           + bl[...])
    nrm = jnp.maximum(jnp.sqrt(jnp.sum(out * out, axis=1, keepdims=True)), 1e-12)
    out = out / nrm
    mu = jnp.mean(out, axis=1, keepdims=True)
    var = jnp.mean((out - mu) ** 2, axis=1, keepdims=True)
    out = (out - mu) / jnp.sqrt(var + 1e-5) * g[...] + b[...]
    out = jnp.maximum(out, 0.0) + hb
    out_ref[...] = out

    hid = jnp.tanh(jnp.dot(out, w1T[...], preferred_element_type=jnp.float32)
                   + b1[...])
    sv = jnp.dot(hid, w2bd[...], preferred_element_type=jnp.float32) + b2[...]
    s_ref[...] = sv
    rows = i * BN + lax.broadcasted_iota(jnp.int32, sv.shape, 0)
    valid = rows < N
    sm = jnp.where(valid, sv, -jnp.inf)
    bm = jnp.max(sm, axis=0, keepdims=True)
    hrows = i * BN + lax.broadcasted_iota(jnp.int32, out.shape, 0)
    out_m = jnp.where(hrows < N, out, 0.0)

    @pl.when(i == 0)
    def _():
        es = jnp.exp(sm - bm)
        m_scr[:1, :ATT_HEADS] = bm
        z_scr[:1, :ATT_HEADS] = jnp.sum(es, axis=0, keepdims=True)
        p_scr[...] = lax.dot_general(es, out_m, (((0,), (0,)), ((), ())),
                                     preferred_element_type=jnp.float32)

    @pl.when(i > 0)
    def _():
        m_old = m_scr[:1, :ATT_HEADS]
        m_new = jnp.maximum(m_old, bm)
        scale = jnp.exp(m_old - m_new)
        es = jnp.exp(sm - m_new)
        m_scr[:1, :ATT_HEADS] = m_new
        z_scr[:1, :ATT_HEADS] = (z_scr[:1, :ATT_HEADS] * scale
                                 + jnp.sum(es, axis=0, keepdims=True))
        p_scr[...] = (p_scr[...] * scale.reshape(ATT_HEADS, 1)
                      + lax.dot_general(es, out_m, (((0,), (0,)), ((), ())),
                                        preferred_element_type=jnp.float32))

    @pl.when(i == GRID_N - 1)
    def _():
        m_ref[...] = m_scr[:1, :ATT_HEADS]
        z_ref[...] = z_scr[:1, :ATT_HEADS]
        p_ref[...] = p_scr[...]


def _tc_layer_scores(agg2, deg2, h, wlT, wrT, bl, g, b, w1T, b1, w2bd, b2):
    full = lambda shape: pl.BlockSpec(shape, lambda i: (0,) * len(shape))
    nblk = N_PAD // BN
    return pl.pallas_call(
        _layer_scores_body,
        grid=(GRID_N,),
        in_specs=[
            pl.BlockSpec((BN, HALF), lambda i: (i, 0)),
            pl.BlockSpec((BN, HALF), lambda i: (nblk + i, 0)),
            pl.BlockSpec((BN, HALF), lambda i: (i, 0)),
            pl.BlockSpec((BN, HALF), lambda i: (nblk + i, 0)),
            pl.BlockSpec((BN, HID), lambda i: (i, 0)),
            full((HID, HID)),
            full((HID, HID)),
            full((1, HID)),
            full((1, HID)),
            full((1, HID)),
            full((HID, ATT_HEADS * ATT_DIM)),
            full((1, ATT_HEADS * ATT_DIM)),
            full((ATT_HEADS * ATT_DIM, ATT_HEADS)),
            full((1, ATT_HEADS)),
        ],
        out_specs=[
            pl.BlockSpec((BN, HID), lambda i: (i, 0)),
            pl.BlockSpec((BN, ATT_HEADS), lambda i: (i, 0)),
            pl.BlockSpec((1, ATT_HEADS), lambda i: (0, 0)),
            pl.BlockSpec((1, ATT_HEADS), lambda i: (0, 0)),
            pl.BlockSpec((ATT_HEADS, HID), lambda i: (0, 0)),
        ],
        out_shape=[
            jax.ShapeDtypeStruct((N, HID), jnp.float32),
            jax.ShapeDtypeStruct((N, ATT_HEADS), jnp.float32),
            jax.ShapeDtypeStruct((1, ATT_HEADS), jnp.float32),
            jax.ShapeDtypeStruct((1, ATT_HEADS), jnp.float32),
            jax.ShapeDtypeStruct((ATT_HEADS, HID), jnp.float32),
        ],
        scratch_shapes=[pltpu.VMEM((8, 128), jnp.float32),
                        pltpu.VMEM((8, 128), jnp.float32),
                        pltpu.VMEM((ATT_HEADS, HID), jnp.float32)],
    )(agg2, agg2, deg2, deg2, h, wlT, wrT, bl, g, b, w1T, b1, w2bd, b2)


def _scores_body(h, w1T, b1, w2bd, b2, s_ref, m_ref, m_scr):
    i = pl.program_id(0)
    hid = jnp.tanh(jnp.dot(h[...], w1T[...], preferred_element_type=jnp.float32)
                   + b1[...])
    s = jnp.dot(hid, w2bd[...], preferred_element_type=jnp.float32) + b2[...]
    s_ref[...] = s
    rows = i * BN + lax.broadcasted_iota(jnp.int32, s.shape, 0)
    sm = jnp.where(rows < N, s, -jnp.inf)
    bm = jnp.max(sm, axis=0, keepdims=True)

    @pl.when(i == 0)
    def _():
        m_scr[:1, :ATT_HEADS] = bm

    @pl.when(i > 0)
    def _():
        m_scr[:1, :ATT_HEADS] = jnp.maximum(m_scr[:1, :ATT_HEADS], bm)

    @pl.when(i == GRID_N - 1)
    def _():
        m_ref[...] = m_scr[:1, :ATT_HEADS]


def _tc_scores(h, w1T, b1, w2bd, b2):
    full = lambda shape: pl.BlockSpec(shape, lambda i: (0,) * len(shape))
    return pl.pallas_call(
        _scores_body,
        grid=(GRID_N,),
        in_specs=[
            pl.BlockSpec((BN, HID), lambda i: (i, 0)),
            full((HID, ATT_HEADS * ATT_DIM)),
            full((1, ATT_HEADS * ATT_DIM)),
            full((ATT_HEADS * ATT_DIM, ATT_HEADS)),
            full((1, ATT_HEADS)),
        ],
        out_specs=[
            pl.BlockSpec((BN, ATT_HEADS), lambda i: (i, 0)),
            pl.BlockSpec((1, ATT_HEADS), lambda i: (0, 0)),
        ],
        out_shape=[
            jax.ShapeDtypeStruct((N, ATT_HEADS), jnp.float32),
            jax.ShapeDtypeStruct((1, ATT_HEADS), jnp.float32),
        ],
        scratch_shapes=[pltpu.VMEM((8, 128), jnp.float32)],
    )(h, w1T, b1, w2bd, b2)


def _pool_body(s, m, h, e_ref, z_ref, p_ref, z_scr, p_scr):
    i = pl.program_id(0)
    e = jnp.exp(s[...] - m[...])
    rows = i * BN + lax.broadcasted_iota(jnp.int32, e.shape, 0)
    e = jnp.where(rows < N, e, 0.0)
    e_ref[...] = e
    zb = jnp.sum(e, axis=0, keepdims=True)
    hb = h[...]
    hrows = i * BN + lax.broadcasted_iota(jnp.int32, hb.shape, 0)
    hb = jnp.where(hrows < N, hb, 0.0)
    pb = lax.dot_general(e, hb, (((0,), (0,)), ((), ())),
                         preferred_element_type=jnp.float32)

    @pl.when(i == 0)
    def _():
        z_scr[:1, :ATT_HEADS] = zb
        p_scr[...] = pb

    @pl.when(i > 0)
    def _():
        z_scr[:1, :ATT_HEADS] = z_scr[:1, :ATT_HEADS] + zb
        p_scr[...] = p_scr[...] + pb

    @pl.when(i == GRID_N - 1)
    def _():
        z_ref[...] = z_scr[:1, :ATT_HEADS]
        p_ref[...] = p_scr[...]


def _tc_pool(s, m, h):
    full = lambda shape: pl.BlockSpec(shape, lambda i: (0,) * len(shape))
    return pl.pallas_call(
        _pool_body,
        grid=(GRID_N,),
        in_specs=[
            pl.BlockSpec((BN, ATT_HEADS), lambda i: (i, 0)),
            full((1, ATT_HEADS)),
            pl.BlockSpec((BN, HID), lambda i: (i, 0)),
        ],
        out_specs=[
            pl.BlockSpec((BN, ATT_HEADS), lambda i: (i, 0)),
            pl.BlockSpec((1, ATT_HEADS), lambda i: (0, 0)),
            pl.BlockSpec((ATT_HEADS, HID), lambda i: (0, 0)),
        ],
        out_shape=[
            jax.ShapeDtypeStruct((N, ATT_HEADS), jnp.float32),
            jax.ShapeDtypeStruct((1, ATT_HEADS), jnp.float32),
            jax.ShapeDtypeStruct((ATT_HEADS, HID), jnp.float32),
        ],
        scratch_shapes=[pltpu.VMEM((8, 128), jnp.float32),
                        pltpu.VMEM((ATT_HEADS, HID), jnp.float32)],
    )(s, m, h)


def _ln_row(t, g, b):
    mu = jnp.mean(t, axis=1, keepdims=True)
    var = jnp.mean((t - mu) ** 2, axis=1, keepdims=True)
    return (t - mu) / jnp.sqrt(var + 1e-5) * g + b


def _final_body(e, m, z, p, wc1T, bc1, g1, b1n, wc2T, bc2, g2, b2n, wc3T, bc3,
                a_ref, probs_ref):
    i = pl.program_id(0)
    a_ref[...] = jnp.exp(e[...] - m[...]) * (1.0 / z[...])

    @pl.when(i == GRID_N - 1)
    def _():
        invz = (1.0 / z[...]).reshape(ATT_HEADS, 1)
        zagg = jnp.mean(p[...] * invz, axis=0, keepdims=True)
        t = jnp.dot(zagg, wc1T[...], preferred_element_type=jnp.float32) + bc1[...]
        t = jnp.maximum(_ln_row(t, g1[...], b1n[...]), 0.0)
        t = jnp.dot(t, wc2T[...], preferred_element_type=jnp.float32) + bc2[...]
        t = jnp.maximum(_ln_row(t, g2[...], b2n[...]), 0.0)
        logits = jnp.dot(t, wc3T[...], preferred_element_type=jnp.float32) + bc3[...]
        mx = jnp.max(logits, axis=1, keepdims=True)
        ex = jnp.exp(logits - mx)
        probs_ref[...] = ex / jnp.sum(ex, axis=1, keepdims=True)


def _tc_final(e, m, z, p, cw):
    full = lambda shape: pl.BlockSpec(shape, lambda i: (0,) * len(shape))
    return pl.pallas_call(
        _final_body,
        grid=(GRID_N,),
        in_specs=[
            pl.BlockSpec((BN, ATT_HEADS), lambda i: (i, 0)),
            full((1, ATT_HEADS)),
            full((1, ATT_HEADS)),
            full((ATT_HEADS, HID)),
            full((HID, CLS)), full((1, CLS)), full((1, CLS)), full((1, CLS)),
            full((CLS, CLS // 2)), full((1, CLS // 2)), full((1, CLS // 2)),
            full((1, CLS // 2)),
            full((CLS // 2, NUM_CLASSES)), full((1, NUM_CLASSES)),
        ],
        out_specs=[
            pl.BlockSpec((BN, ATT_HEADS), lambda i: (i, 0)),
            pl.BlockSpec((1, NUM_CLASSES), lambda i: (0, 0)),
        ],
        out_shape=[
            jax.ShapeDtypeStruct((N, ATT_HEADS), jnp.float32),
            jax.ShapeDtypeStruct((1, NUM_CLASSES), jnp.float32),
        ],
    )(e, m, z, p, *cw)


def kernel(x, edge_index, params):
    src = edge_index[0]
    dst = edge_index[1]
    srcg = jnp.concatenate([src * 2, src * 2 + 1])
    zeros_hbm = jnp.zeros((N_PAD, HALF), jnp.float32)
    ones_hbm = jnp.ones((EB, HALF), jnp.float32)

    att = params['att']
    w1T = jnp.concatenate([ap['W1'] for ap in att], axis=0).T
    b1 = jnp.concatenate([ap['b1'] for ap in att]).reshape(1, -1)
    w2bd = jnp.zeros((ATT_HEADS * ATT_DIM, ATT_HEADS), jnp.float32)
    for hh, ap in enumerate(att):
        w2bd = w2bd.at[hh * ATT_DIM:(hh + 1) * ATT_DIM, hh].set(ap['W2'][0])
    b2 = jnp.stack([ap['b2'][0] for ap in att]).reshape(1, ATT_HEADS)

    lp0, lp1 = params['layers']
    agg2, deg2 = _sc_segment_sum(x.reshape(2 * N, HALF), srcg, dst, zeros_hbm,
                                 ones_hbm)
    h = _tc_layer(agg2, deg2, x, lp0['W_l'].T, lp0['W_r'].T,
                  lp0['b_l'].reshape(1, HID), lp0['ln_g'].reshape(1, HID),
                  lp0['ln_b'].reshape(1, HID))
    agg2 = _sc_segment_sum(h.reshape(2 * N, HALF), srcg, dst, zeros_hbm)
    h, s, m, z, p = _tc_layer_scores(agg2, deg2, h, lp1['W_l'].T, lp1['W_r'].T,
                                     lp1['b_l'].reshape(1, HID),
                                     lp1['ln_g'].reshape(1, HID),
                                     lp1['ln_b'].reshape(1, HID),
                                     w1T, b1, w2bd, b2)

    c = params['cls']
    cw = (c['W1'].T, c['b1'].reshape(1, -1), c['g1'].reshape(1, -1),
          c['b1n'].reshape(1, -1),
          c['W2'].T, c['b2'].reshape(1, -1), c['g2'].reshape(1, -1),
          c['b2n'].reshape(1, -1),
          c['W3'].T, c['b3'].reshape(1, -1))
    a, probs = _tc_final(s, m, z, p, cw)
    return probs[0], a


# speedup vs baseline: 1.2859x; 1.0199x over previous
import functools

import jax
import jax.numpy as jnp
from jax import lax
from jax.experimental import pallas as pl
from jax.experimental.pallas import tpu as pltpu
from jax.experimental.pallas import tpu_sc as plsc

N = 10000
E = 160000
D = 256
HID = 256
ATT_DIM = 128
ATT_HEADS = 4
CLS = 128
NUM_CLASSES = 7

HALF = D // 2
EB = 128
NB_TOTAL = E // EB
NSUB = 16
TB_MAX = 80
TB_LAST = NB_TOTAL - TB_MAX * (NSUB - 1)
TB_LAST_PAD = 56
NB_PADDED = TB_MAX * (NSUB - 1) + TB_LAST_PAD
N_PAD = 10240
ROWS_PER_TILE = N_PAD // NSUB

EHALF = E // 2
DB_TOTAL = EHALF // EB
DB_BASE = DB_TOTAL // NSUB
DB_REM = DB_TOTAL - DB_BASE * NSUB

BN = 2048
GRID_N = (N + BN - 1) // BN


def _sc_segment_sum(h2, srcg, dst, zeros_hbm, ones_hbm=None):
    with_deg = ones_hbm is not None
    mesh = plsc.VectorSubcoreMesh(core_axis_name="c", subcore_axis_name="s")

    @functools.partial(
        pl.kernel,
        out_type=([jax.ShapeDtypeStruct((2 * N_PAD, HALF), jnp.float32)] * 2
                  if with_deg else
                  jax.ShapeDtypeStruct((2 * N_PAD, HALF), jnp.float32)),
        mesh=mesh,
        scratch_types=[
            pltpu.VMEM_SHARED((N_PAD, HALF), jnp.float32),
            pltpu.VMEM((TB_MAX * EB,), jnp.int32),
            pltpu.VMEM((2, EB), jnp.int32),
            pltpu.VMEM((2, EB, HALF), jnp.float32),
            pltpu.SemaphoreType.DMA,
            pltpu.SemaphoreType.DMA,
            pltpu.SemaphoreType.DMA,
            pltpu.SemaphoreType.DMA,
        ],
    )
    def k(*args):
        if with_deg:
            (h2_hbm, srcg_hbm, dst_hbm, zeros_hbm_, ones_hbm_, agg_out,
             deg_out, agg_sh, idx_all, dst2_v, rows2,
             sem0, sem1, dem0, dem1) = args
        else:
            (h2_hbm, srcg_hbm, dst_hbm, zeros_hbm_, agg_out,
             agg_sh, idx_all, dst2_v, rows2, sem0, sem1, dem0, dem1) = args
        c = lax.axis_index("c")
        s = lax.axis_index("s")
        r0 = s * ROWS_PER_TILE
        pltpu.sync_copy(zeros_hbm_.at[pl.ds(r0, ROWS_PER_TILE)],
                        agg_sh.at[pl.ds(r0, ROWS_PER_TILE)])

        b0 = s * TB_MAX

        @pl.when(s < NSUB - 1)
        def _():
            pltpu.sync_copy(srcg_hbm.at[pl.ds(c * E + b0 * EB, TB_MAX * EB)],
                            idx_all)

        @pl.when(s == NSUB - 1)
        def _():
            pltpu.sync_copy(srcg_hbm.at[pl.ds(c * E + b0 * EB, TB_LAST * EB)],
                            idx_all.at[pl.ds(0, TB_LAST * EB)])

        plsc.subcore_barrier()

        nh = jnp.where(s < NSUB - 1, TB_MAX // 2, TB_LAST // 2)

        def gather(b, buf, sem):
            pltpu.async_copy(h2_hbm.at[idx_all.at[pl.ds(b * EB, EB)]],
                             rows2.at[buf], sem)

        def gwait(buf, sem):
            pltpu.make_async_copy(h2_hbm.at[pl.ds(0, EB)], rows2.at[buf], sem).wait()

        def dload(b, buf, sem):
            pltpu.async_copy(dst_hbm.at[pl.ds((b0 + b) * EB, EB)],
                             dst2_v.at[buf], sem)

        def dwait(buf, sem):
            pltpu.make_async_copy(dst_hbm.at[pl.ds(0, EB)], dst2_v.at[buf],
                                  sem).wait()

        gather(0, 0, sem0)
        gather(1, 1, sem1)
        dload(0, 0, dem0)
        dload(1, 1, dem1)

        def body(j, carry):
            b_even = 2 * j

            gwait(0, sem0)
            dwait(0, dem0)
            pltpu.sync_copy(rows2.at[0], agg_sh.at[dst2_v.at[0]], add=True)

            @pl.when(j < nh - 1)
            def _():
                gather(b_even + 2, 0, sem0)
                dload(b_even + 2, 0, dem0)

            gwait(1, sem1)
            dwait(1, dem1)
            pltpu.sync_copy(rows2.at[1], agg_sh.at[dst2_v.at[1]], add=True)

            @pl.when(j < nh - 1)
            def _():
                gather(b_even + 3, 1, sem1)
                dload(b_even + 3, 1, dem1)

            return carry

        lax.fori_loop(0, nh, body, 0)

        plsc.subcore_barrier()
        pltpu.sync_copy(agg_sh.at[pl.ds(r0, ROWS_PER_TILE)],
                        agg_out.at[pl.ds(c * N_PAD + r0, ROWS_PER_TILE)])

        if with_deg:
            plsc.subcore_barrier()
            pltpu.sync_copy(zeros_hbm_.at[pl.ds(r0, ROWS_PER_TILE)],
                            agg_sh.at[pl.ds(r0, ROWS_PER_TILE)])
            pltpu.sync_copy(ones_hbm_, rows2.at[0])
            plsc.subcore_barrier()

            dnb = jnp.where(s < 1, DB_BASE + 1, DB_BASE)
            dbase = c * DB_TOTAL + s * DB_BASE + jnp.minimum(s, 1)

            def ddload(b, buf, sem):
                pltpu.async_copy(dst_hbm.at[pl.ds((dbase + b) * EB, EB)],
                                 dst2_v.at[buf], sem)

            def ddwait(buf, sem):
                pltpu.make_async_copy(dst_hbm.at[pl.ds(0, EB)], dst2_v.at[buf],
                                      sem).wait()

            ddload(0, 0, dem0)
            ddload(1, 1, dem1)

            def dhalf(b, buf, sem):
                @pl.when(b < dnb)
                def _():
                    ddwait(buf, sem)
                    pltpu.sync_copy(rows2.at[0], agg_sh.at[dst2_v.at[buf]],
                                    add=True)

                    @pl.when(b + 2 < dnb)
                    def _():
                        ddload(b + 2, buf, sem)

            def dbody(j, carry):
                dhalf(2 * j, 0, dem0)
                dhalf(2 * j + 1, 1, dem1)
                return carry

            lax.fori_loop(0, (DB_BASE + 2) // 2, dbody, 0)

            plsc.subcore_barrier()
            pltpu.sync_copy(agg_sh.at[pl.ds(r0, ROWS_PER_TILE)],
                            deg_out.at[pl.ds(c * N_PAD + r0, ROWS_PER_TILE)])

    if with_deg:
        return k(h2, srcg, dst, zeros_hbm, ones_hbm)
    return k(h2, srcg, dst, zeros_hbm)


def _sc_degree(dst, zeros_hbm, ones_hbm):
    mesh = plsc.VectorSubcoreMesh(core_axis_name="c", subcore_axis_name="s")

    @functools.partial(
        pl.kernel,
        out_type=jax.ShapeDtypeStruct((2 * N_PAD, HALF), jnp.float32),
        mesh=mesh,
        scratch_types=[
            pltpu.VMEM_SHARED((N_PAD, HALF), jnp.float32),
            pltpu.VMEM((2, EB), jnp.int32),
            pltpu.VMEM((EB, HALF), jnp.float32),
            pltpu.SemaphoreType.DMA,
            pltpu.SemaphoreType.DMA,
        ],
    )
    def k(dst_hbm, zeros_hbm_, ones_hbm_, deg_out, deg_sh, dst2_v, ones_v,
          dem0, dem1):
        c = lax.axis_index("c")
        s = lax.axis_index("s")
        r0 = s * ROWS_PER_TILE
        pltpu.sync_copy(zeros_hbm_.at[pl.ds(r0, ROWS_PER_TILE)],
                        deg_sh.at[pl.ds(r0, ROWS_PER_TILE)])
        pltpu.sync_copy(ones_hbm_, ones_v)
        plsc.subcore_barrier()

        nb = jnp.where(s < 1, DB_BASE + 1, DB_BASE)
        dbase = c * DB_TOTAL + s * DB_BASE + jnp.minimum(s, 1)

        def dload(b, buf, sem):
            pltpu.async_copy(dst_hbm.at[pl.ds((dbase + b) * EB, EB)],
                             dst2_v.at[buf], sem)

        def dwait(buf, sem):
            pltpu.make_async_copy(dst_hbm.at[pl.ds(0, EB)], dst2_v.at[buf],
                                  sem).wait()

        dload(0, 0, dem0)
        dload(1, 1, dem1)

        def half(j, b, buf, sem):
            @pl.when(b < nb)
            def _():
                dwait(buf, sem)
                pltpu.sync_copy(ones_v, deg_sh.at[dst2_v.at[buf]], add=True)

                @pl.when(b + 2 < nb)
                def _():
                    dload(b + 2, buf, sem)

        def body(j, carry):
            half(j, 2 * j, 0, dem0)
            half(j, 2 * j + 1, 1, dem1)
            return carry

        lax.fori_loop(0, (DB_BASE + 2) // 2, body, 0)

        plsc.subcore_barrier()
        pltpu.sync_copy(deg_sh.at[pl.ds(r0, ROWS_PER_TILE)],
                        deg_out.at[pl.ds(c * N_PAD + r0, ROWS_PER_TILE)])

    return k(dst, zeros_hbm, ones_hbm)


def _layer_body(agg_lo, agg_hi, dega, degb, h, wlT, wrT, bl, g, b, out_ref):
    deg = dega[:, :1] + degb[:, :1]
    inv = 1.0 / jnp.maximum(deg, 1.0)
    mean = jnp.concatenate([agg_lo[...], agg_hi[...]], axis=1) * inv
    hb = h[...]
    out = (jnp.dot(mean, wlT[...], preferred_element_type=jnp.float32)
           + jnp.dot(hb, wrT[...], preferred_element_type=jnp.float32)
           + bl[...])
    nrm = jnp.maximum(jnp.sqrt(jnp.sum(out * out, axis=1, keepdims=True)), 1e-12)
    out = out / nrm
    mu = jnp.mean(out, axis=1, keepdims=True)
    var = jnp.mean((out - mu) ** 2, axis=1, keepdims=True)
    out = (out - mu) / jnp.sqrt(var + 1e-5) * g[...] + b[...]
    out = jnp.maximum(out, 0.0)
    out_ref[...] = out + hb


def _tc_layer(agg2, deg2, h, wlT, wrT, bl, g, b):
    full = lambda shape: pl.BlockSpec(shape, lambda i: (0,) * len(shape))
    nblk = N_PAD // BN
    return pl.pallas_call(
        _layer_body,
        grid=(GRID_N,),
        in_specs=[
            pl.BlockSpec((BN, HALF), lambda i: (i, 0)),
            pl.BlockSpec((BN, HALF), lambda i: (nblk + i, 0)),
            pl.BlockSpec((BN, HALF), lambda i: (i, 0)),
            pl.BlockSpec((BN, HALF), lambda i: (nblk + i, 0)),
            pl.BlockSpec((BN, HID), lambda i: (i, 0)),
            full((HID, HID)),
            full((HID, HID)),
            full((1, HID)),
            full((1, HID)),
            full((1, HID)),
        ],
        out_specs=pl.BlockSpec((BN, HID), lambda i: (i, 0)),
        out_shape=jax.ShapeDtypeStruct((N, HID), jnp.float32),
    )(agg2, agg2, deg2, deg2, h, wlT, wrT, bl, g, b)


def _layer_scores_body(agg_lo, agg_hi, dega, degb, h, wlT, wrT, bl, g, b,
                       w1T, b1, w2bd, b2, out_ref, s_ref, m_ref, z_ref, p_ref,
                       m_scr, z_scr, p_scr):
    i = pl.program_id(0)
    deg = dega[:, :1] + degb[:, :1]
    inv = 1.0 / jnp.maximum(deg, 1.0)
    mean = jnp.concatenate([agg_lo[...], agg_hi[...]], axis=1) * inv
    hb = h[...]
    out = (jnp.dot(mean, wlT[...], preferred_element_type=jnp.float32)
           + jnp.dot(hb, wrT[...], preferred_element_type=jnp.float32)
           + bl[...])
    nrm = jnp.maximum(jnp.sqrt(jnp.sum(out * out, axis=1, keepdims=True)), 1e-12)
    out = out / nrm
    mu = jnp.mean(out, axis=1, keepdims=True)
    var = jnp.mean((out - mu) ** 2, axis=1, keepdims=True)
    out = (out - mu) / jnp.sqrt(var + 1e-5) * g[...] + b[...]
    out = jnp.maximum(out, 0.0) + hb
    out_ref[...] = out

    hid = jnp.tanh(jnp.dot(out, w1T[...], preferred_element_type=jnp.float32)
                   + b1[...])
    sv = jnp.dot(hid, w2bd[...], preferred_element_type=jnp.float32) + b2[...]
    s_ref[...] = sv
    rows = i * BN + lax.broadcasted_iota(jnp.int32, sv.shape, 0)
    valid = rows < N
    sm = jnp.where(valid, sv, -jnp.inf)
    bm = jnp.max(sm, axis=0, keepdims=True)
    hrows = i * BN + lax.broadcasted_iota(jnp.int32, out.shape, 0)
    out_m = jnp.where(hrows < N, out, 0.0)

    @pl.when(i == 0)
    def _():
        es = jnp.exp(sm - bm)
        m_scr[:1, :ATT_HEADS] = bm
        z_scr[:1, :ATT_HEADS] = jnp.sum(es, axis=0, keepdims=True)
        p_scr[...] = lax.dot_general(es, out_m, (((0,), (0,)), ((), ())),
                                     preferred_element_type=jnp.float32)

    @pl.when(i > 0)
    def _():
        m_old = m_scr[:1, :ATT_HEADS]
        m_new = jnp.maximum(m_old, bm)
        scale = jnp.exp(m_old - m_new)
        es = jnp.exp(sm - m_new)
        m_scr[:1, :ATT_HEADS] = m_new
        z_scr[:1, :ATT_HEADS] = (z_scr[:1, :ATT_HEADS] * scale
                                 + jnp.sum(es, axis=0, keepdims=True))
        p_scr[...] = (p_scr[...] * scale.reshape(ATT_HEADS, 1)
                      + lax.dot_general(es, out_m, (((0,), (0,)), ((), ())),
                                        preferred_element_type=jnp.float32))

    @pl.when(i == GRID_N - 1)
    def _():
        m_ref[...] = m_scr[:1, :ATT_HEADS]
        z_ref[...] = z_scr[:1, :ATT_HEADS]
        p_ref[...] = p_scr[...]


def _tc_layer_scores(agg2, deg2, h, wlT, wrT, bl, g, b, w1T, b1, w2bd, b2):
    full = lambda shape: pl.BlockSpec(shape, lambda i: (0,) * len(shape))
    nblk = N_PAD // BN
    return pl.pallas_call(
        _layer_scores_body,
        grid=(GRID_N,),
        in_specs=[
            pl.BlockSpec((BN, HALF), lambda i: (i, 0)),
            pl.BlockSpec((BN, HALF), lambda i: (nblk + i, 0)),
            pl.BlockSpec((BN, HALF), lambda i: (i, 0)),
            pl.BlockSpec((BN, HALF), lambda i: (nblk + i, 0)),
            pl.BlockSpec((BN, HID), lambda i: (i, 0)),
            full((HID, HID)),
            full((HID, HID)),
            full((1, HID)),
            full((1, HID)),
            full((1, HID)),
            full((HID, ATT_HEADS * ATT_DIM)),
            full((1, ATT_HEADS * ATT_DIM)),
            full((ATT_HEADS * ATT_DIM, ATT_HEADS)),
            full((1, ATT_HEADS)),
        ],
        out_specs=[
            pl.BlockSpec((BN, HID), lambda i: (i, 0)),
            pl.BlockSpec((BN, ATT_HEADS), lambda i: (i, 0)),
            pl.BlockSpec((1, ATT_HEADS), lambda i: (0, 0)),
            pl.BlockSpec((1, ATT_HEADS), lambda i: (0, 0)),
            pl.BlockSpec((ATT_HEADS, HID), lambda i: (0, 0)),
        ],
        out_shape=[
            jax.ShapeDtypeStruct((N, HID), jnp.float32),
            jax.ShapeDtypeStruct((N, ATT_HEADS), jnp.float32),
            jax.ShapeDtypeStruct((1, ATT_HEADS), jnp.float32),
            jax.ShapeDtypeStruct((1, ATT_HEADS), jnp.float32),
            jax.ShapeDtypeStruct((ATT_HEADS, HID), jnp.float32),
        ],
        scratch_shapes=[pltpu.VMEM((8, 128), jnp.float32),
                        pltpu.VMEM((8, 128), jnp.float32),
                        pltpu.VMEM((ATT_HEADS, HID), jnp.float32)],
    )(agg2, agg2, deg2, deg2, h, wlT, wrT, bl, g, b, w1T, b1, w2bd, b2)


def _scores_body(h, w1T, b1, w2bd, b2, s_ref, m_ref, m_scr):
    i = pl.program_id(0)
    hid = jnp.tanh(jnp.dot(h[...], w1T[...], preferred_element_type=jnp.float32)
                   + b1[...])
    s = jnp.dot(hid, w2bd[...], preferred_element_type=jnp.float32) + b2[...]
    s_ref[...] = s
    rows = i * BN + lax.broadcasted_iota(jnp.int32, s.shape, 0)
    sm = jnp.where(rows < N, s, -jnp.inf)
    bm = jnp.max(sm, axis=0, keepdims=True)

    @pl.when(i == 0)
    def _():
        m_scr[:1, :ATT_HEADS] = bm

    @pl.when(i > 0)
    def _():
        m_scr[:1, :ATT_HEADS] = jnp.maximum(m_scr[:1, :ATT_HEADS], bm)

    @pl.when(i == GRID_N - 1)
    def _():
        m_ref[...] = m_scr[:1, :ATT_HEADS]


def _tc_scores(h, w1T, b1, w2bd, b2):
    full = lambda shape: pl.BlockSpec(shape, lambda i: (0,) * len(shape))
    return pl.pallas_call(
        _scores_body,
        grid=(GRID_N,),
        in_specs=[
            pl.BlockSpec((BN, HID), lambda i: (i, 0)),
            full((HID, ATT_HEADS * ATT_DIM)),
            full((1, ATT_HEADS * ATT_DIM)),
            full((ATT_HEADS * ATT_DIM, ATT_HEADS)),
            full((1, ATT_HEADS)),
        ],
        out_specs=[
            pl.BlockSpec((BN, ATT_HEADS), lambda i: (i, 0)),
            pl.BlockSpec((1, ATT_HEADS), lambda i: (0, 0)),
        ],
        out_shape=[
            jax.ShapeDtypeStruct((N, ATT_HEADS), jnp.float32),
            jax.ShapeDtypeStruct((1, ATT_HEADS), jnp.float32),
        ],
        scratch_shapes=[pltpu.VMEM((8, 128), jnp.float32)],
    )(h, w1T, b1, w2bd, b2)


def _pool_body(s, m, h, e_ref, z_ref, p_ref, z_scr, p_scr):
    i = pl.program_id(0)
    e = jnp.exp(s[...] - m[...])
    rows = i * BN + lax.broadcasted_iota(jnp.int32, e.shape, 0)
    e = jnp.where(rows < N, e, 0.0)
    e_ref[...] = e
    zb = jnp.sum(e, axis=0, keepdims=True)
    hb = h[...]
    hrows = i * BN + lax.broadcasted_iota(jnp.int32, hb.shape, 0)
    hb = jnp.where(hrows < N, hb, 0.0)
    pb = lax.dot_general(e, hb, (((0,), (0,)), ((), ())),
                         preferred_element_type=jnp.float32)

    @pl.when(i == 0)
    def _():
        z_scr[:1, :ATT_HEADS] = zb
        p_scr[...] = pb

    @pl.when(i > 0)
    def _():
        z_scr[:1, :ATT_HEADS] = z_scr[:1, :ATT_HEADS] + zb
        p_scr[...] = p_scr[...] + pb

    @pl.when(i == GRID_N - 1)
    def _():
        z_ref[...] = z_scr[:1, :ATT_HEADS]
        p_ref[...] = p_scr[...]


def _tc_pool(s, m, h):
    full = lambda shape: pl.BlockSpec(shape, lambda i: (0,) * len(shape))
    return pl.pallas_call(
        _pool_body,
        grid=(GRID_N,),
        in_specs=[
            pl.BlockSpec((BN, ATT_HEADS), lambda i: (i, 0)),
            full((1, ATT_HEADS)),
            pl.BlockSpec((BN, HID), lambda i: (i, 0)),
        ],
        out_specs=[
            pl.BlockSpec((BN, ATT_HEADS), lambda i: (i, 0)),
            pl.BlockSpec((1, ATT_HEADS), lambda i: (0, 0)),
            pl.BlockSpec((ATT_HEADS, HID), lambda i: (0, 0)),
        ],
        out_shape=[
            jax.ShapeDtypeStruct((N, ATT_HEADS), jnp.float32),
            jax.ShapeDtypeStruct((1, ATT_HEADS), jnp.float32),
            jax.ShapeDtypeStruct((ATT_HEADS, HID), jnp.float32),
        ],
        scratch_shapes=[pltpu.VMEM((8, 128), jnp.float32),
                        pltpu.VMEM((ATT_HEADS, HID), jnp.float32)],
    )(s, m, h)


def _ln_row(t, g, b):
    mu = jnp.mean(t, axis=1, keepdims=True)
    var = jnp.mean((t - mu) ** 2, axis=1, keepdims=True)
    return (t - mu) / jnp.sqrt(var + 1e-5) * g + b


def _final_body(e, m, z, p, wc1T, bc1, g1, b1n, wc2T, bc2, g2, b2n, wc3T, bc3,
                a_ref, probs_ref):
    i = pl.program_id(0)
    a_ref[...] = jnp.exp(e[...] - m[...]) * (1.0 / z[...])

    @pl.when(i == GRID_N - 1)
    def _():
        invz = (1.0 / z[...]).reshape(ATT_HEADS, 1)
        zagg = jnp.mean(p[...] * invz, axis=0, keepdims=True)
        t = jnp.dot(zagg, wc1T[...], preferred_element_type=jnp.float32) + bc1[...]
        t = jnp.maximum(_ln_row(t, g1[...], b1n[...]), 0.0)
        t = jnp.dot(t, wc2T[...], preferred_element_type=jnp.float32) + bc2[...]
        t = jnp.maximum(_ln_row(t, g2[...], b2n[...]), 0.0)
        logits = jnp.dot(t, wc3T[...], preferred_element_type=jnp.float32) + bc3[...]
        mx = jnp.max(logits, axis=1, keepdims=True)
        ex = jnp.exp(logits - mx)
        probs_ref[...] = ex / jnp.sum(ex, axis=1, keepdims=True)


def _tc_final(e, m, z, p, cw):
    full = lambda shape: pl.BlockSpec(shape, lambda i: (0,) * len(shape))
    return pl.pallas_call(
        _final_body,
        grid=(GRID_N,),
        in_specs=[
            pl.BlockSpec((BN, ATT_HEADS), lambda i: (i, 0)),
            full((1, ATT_HEADS)),
            full((1, ATT_HEADS)),
            full((ATT_HEADS, HID)),
            full((HID, CLS)), full((1, CLS)), full((1, CLS)), full((1, CLS)),
            full((CLS, CLS // 2)), full((1, CLS // 2)), full((1, CLS // 2)),
            full((1, CLS // 2)),
            full((CLS // 2, NUM_CLASSES)), full((1, NUM_CLASSES)),
        ],
        out_specs=[
            pl.BlockSpec((BN, ATT_HEADS), lambda i: (i, 0)),
            pl.BlockSpec((1, NUM_CLASSES), lambda i: (0, 0)),
        ],
        out_shape=[
            jax.ShapeDtypeStruct((N, ATT_HEADS), jnp.float32),
            jax.ShapeDtypeStruct((1, NUM_CLASSES), jnp.float32),
        ],
    )(e, m, z, p, *cw)


def kernel(x, edge_index, params):
    src = edge_index[0]
    dst = edge_index[1]
    srcg = jnp.concatenate([src * 2, src * 2 + 1])
    zeros_hbm = jnp.zeros((N_PAD, HALF), jnp.float32)
    ones_hbm = jnp.ones((EB, HALF), jnp.float32)

    att = params['att']
    w1T = jnp.concatenate([ap['W1'] for ap in att], axis=0).T
    b1 = jnp.concatenate([ap['b1'] for ap in att]).reshape(1, -1)
    w2bd = jnp.zeros((ATT_HEADS * ATT_DIM, ATT_HEADS), jnp.float32)
    for hh, ap in enumerate(att):
        w2bd = w2bd.at[hh * ATT_DIM:(hh + 1) * ATT_DIM, hh].set(ap['W2'][0])
    b2 = jnp.stack([ap['b2'][0] for ap in att]).reshape(1, ATT_HEADS)

    lp0, lp1 = params['layers']
    agg2, deg2 = _sc_segment_sum(x.reshape(2 * N, HALF), srcg, dst, zeros_hbm,
                                 ones_hbm)
    h = _tc_layer(agg2, deg2, x, lp0['W_l'].T, lp0['W_r'].T,
                  lp0['b_l'].reshape(1, HID), lp0['ln_g'].reshape(1, HID),
                  lp0['ln_b'].reshape(1, HID))
    agg2 = _sc_segment_sum(h.reshape(2 * N, HALF), srcg, dst, zeros_hbm)
    h, s, m, z, p = _tc_layer_scores(agg2, deg2, h, lp1['W_l'].T, lp1['W_r'].T,
                                     lp1['b_l'].reshape(1, HID),
                                     lp1['ln_g'].reshape(1, HID),
                                     lp1['ln_b'].reshape(1, HID),
                                     w1T, b1, w2bd, b2)

    c = params['cls']
    cw = (c['W1'].T, c['b1'].reshape(1, -1), c['g1'].reshape(1, -1),
          c['b1n'].reshape(1, -1),
          c['W2'].T, c['b2'].reshape(1, -1), c['g2'].reshape(1, -1),
          c['b2n'].reshape(1, -1),
          c['W3'].T, c['b3'].reshape(1, -1))
    a, probs = _tc_final(s, m, z, p, cw)
    return probs[0], a
